# baseline (speedup 1.0000x reference)
import functools
import jax
import jax.numpy as jnp
from jax import lax
from jax.experimental import pallas as pl
from jax.experimental.pallas import tpu as pltpu

N = 10000
E = 320000
D = 128

_PREC = lax.Precision.HIGHEST


def _silu(v):
    return v * jax.nn.sigmoid(v)


def _pre_body(h_ref, Wr_ref, Wc_ref, We2_ref, Wc1_ref, be2_ref, bc1_ref,
              P_ref, Q_ref, Wec_ref, bec_ref):
    h = h_ref[...]
    P_ref[...] = jnp.dot(h, Wr_ref[...], precision=_PREC)
    Q_ref[...] = jnp.dot(h, Wc_ref[...], precision=_PREC)
    Wec_ref[...] = jnp.dot(We2_ref[...], Wc1_ref[...], precision=_PREC)
    bec_ref[...] = jnp.dot(be2_ref[...], Wc1_ref[...], precision=_PREC) + bc1_ref[...]


def _node_pre(h, Wr, Wc, We2, Wc1, be2, bc1):
    NB = 1000
    grid = (N // NB,)
    return pl.pallas_call(
        _pre_body,
        grid=grid,
        in_specs=[
            pl.BlockSpec((NB, D), lambda i: (i, 0)),
            pl.BlockSpec((D, D), lambda i: (0, 0)),
            pl.BlockSpec((D, D), lambda i: (0, 0)),
            pl.BlockSpec((D, D), lambda i: (0, 0)),
            pl.BlockSpec((D, D), lambda i: (0, 0)),
            pl.BlockSpec((1, D), lambda i: (0, 0)),
            pl.BlockSpec((1, D), lambda i: (0, 0)),
        ],
        out_specs=[
            pl.BlockSpec((NB, D), lambda i: (i, 0)),
            pl.BlockSpec((NB, D), lambda i: (i, 0)),
            pl.BlockSpec((D, D), lambda i: (0, 0)),
            pl.BlockSpec((1, D), lambda i: (0, 0)),
        ],
        out_shape=[
            jax.ShapeDtypeStruct((N, D), jnp.float32),
            jax.ShapeDtypeStruct((N, D), jnp.float32),
            jax.ShapeDtypeStruct((D, D), jnp.float32),
            jax.ShapeDtypeStruct((1, D), jnp.float32),
        ],
    )(h, Wr, Wc, We2, Wc1, be2, bc1)


def _edge_body(G1_ref, G2_ref, xr_ref, xc_ref, ea_ref,
               Ws_ref, Wa_ref, be1_ref, Wec_ref, bec_ref, Wc2_ref, bc2_ref,
               s_ref, fc_ref):
    rij = xr_ref[...] - xc_ref[...]
    scalar = jnp.sum(rij * rij, axis=1, keepdims=True)
    u = (G1_ref[...] + G2_ref[...]
         + scalar * Ws_ref[...]
         + jnp.dot(ea_ref[...], Wa_ref[...], precision=_PREC)
         + be1_ref[...])
    s = _silu(u)
    s_ref[...] = s
    t = jnp.dot(s, Wec_ref[...], precision=_PREC) + bec_ref[...]
    cm = jnp.dot(_silu(t), Wc2_ref[...], precision=_PREC) + bc2_ref[...]
    ones = (lax.broadcasted_iota(jnp.int32, rij.shape, 1) == 3).astype(jnp.float32)
    fc_ref[...] = rij * cm[:, 0:1] + ones


def _edge_mlp(G1, G2, xr, xc, ea, Ws, Wa, be1, Wec, bec, Wc2, bc2):
    EB = 4000
    grid = (E // EB,)
    return pl.pallas_call(
        _edge_body,
        grid=grid,
        in_specs=[
            pl.BlockSpec((EB, D), lambda i: (i, 0)),
            pl.BlockSpec((EB, D), lambda i: (i, 0)),
            pl.BlockSpec((EB, 8), lambda i: (i, 0)),
            pl.BlockSpec((EB, 8), lambda i: (i, 0)),
            pl.BlockSpec((EB, 8), lambda i: (i, 0)),
            pl.BlockSpec((1, D), lambda i: (0, 0)),
            pl.BlockSpec((8, D), lambda i: (0, 0)),
            pl.BlockSpec((1, D), lambda i: (0, 0)),
            pl.BlockSpec((D, D), lambda i: (0, 0)),
            pl.BlockSpec((1, D), lambda i: (0, 0)),
            pl.BlockSpec((D, 128), lambda i: (0, 0)),
            pl.BlockSpec((1, 128), lambda i: (0, 0)),
        ],
        out_specs=[
            pl.BlockSpec((EB, D), lambda i: (i, 0)),
            pl.BlockSpec((EB, 8), lambda i: (i, 0)),
        ],
        out_shape=[
            jax.ShapeDtypeStruct((E, D), jnp.float32),
            jax.ShapeDtypeStruct((E, 8), jnp.float32),
        ],
    )(G1, G2, xr, xc, ea, Ws, Wa, be1, Wec, bec, Wc2, bc2)


def _post_body(h_ref, S_ref, FC_ref, xp_ref,
               We2_ref, be2_ref, Wn1h_ref, Wn1t_ref, bn1_ref, Wn2_ref, bn2_ref,
               hn_ref, xf_ref):
    h = h_ref[...]
    FC = FC_ref[...]
    cnt = FC[:, 3:4]
    tot = jnp.dot(S_ref[...], We2_ref[...], precision=_PREC) + cnt * be2_ref[...]
    a = (jnp.dot(h, Wn1h_ref[...], precision=_PREC)
         + jnp.dot(tot, Wn1t_ref[...], precision=_PREC) + bn1_ref[...])
    hn_ref[...] = h + jnp.dot(_silu(a), Wn2_ref[...], precision=_PREC) + bn2_ref[...]
    tot_f = jnp.clip(FC / jnp.maximum(cnt, 1.0), -100.0, 100.0)
    xf_ref[...] = xp_ref[...] + tot_f


def _node_post(h, S, FC, xp, We2, be2, Wn1h, Wn1t, bn1, Wn2, bn2):
    NB = 1000
    grid = (N // NB,)
    return pl.pallas_call(
        _post_body,
        grid=grid,
        in_specs=[
            pl.BlockSpec((NB, D), lambda i: (i, 0)),
            pl.BlockSpec((NB, D), lambda i: (i, 0)),
            pl.BlockSpec((NB, 8), lambda i: (i, 0)),
            pl.BlockSpec((NB, 8), lambda i: (i, 0)),
            pl.BlockSpec((D, D), lambda i: (0, 0)),
            pl.BlockSpec((1, D), lambda i: (0, 0)),
            pl.BlockSpec((D, D), lambda i: (0, 0)),
            pl.BlockSpec((D, D), lambda i: (0, 0)),
            pl.BlockSpec((1, D), lambda i: (0, 0)),
            pl.BlockSpec((D, D), lambda i: (0, 0)),
            pl.BlockSpec((1, D), lambda i: (0, 0)),
        ],
        out_specs=[
            pl.BlockSpec((NB, D), lambda i: (i, 0)),
            pl.BlockSpec((NB, 8), lambda i: (i, 0)),
        ],
        out_shape=[
            jax.ShapeDtypeStruct((N, D), jnp.float32),
            jax.ShapeDtypeStruct((N, 8), jnp.float32),
        ],
    )(h, S, FC, xp, We2, be2, Wn1h, Wn1t, bn1, Wn2, bn2)


@jax.jit
def kernel(x, h, edge_index, edge_attr,
           W_e1, b_e1, W_e2, b_e2,
           W_c1, b_c1, W_c2, b_c2,
           W_n1, b_n1, W_n2, b_n2):
    row = edge_index[0]
    col = edge_index[1]
    Ws = W_e1[0:1]
    Wr = W_e1[1:1 + D]
    Wc = W_e1[1 + D:1 + 2 * D]
    Wa = jnp.zeros((8, D), jnp.float32).at[:5].set(W_e1[1 + 2 * D:])
    be1 = b_e1[None, :]
    be2 = b_e2[None, :]
    bc1 = b_c1[None, :]
    bc2 = jnp.zeros((1, 128), jnp.float32).at[0, 0].set(b_c2[0])
    Wc2 = jnp.zeros((D, 128), jnp.float32).at[:, 0:1].set(W_c2)
    Wn1h = W_n1[:D]
    Wn1t = W_n1[D:]
    bn1 = b_n1[None, :]
    bn2 = b_n2[None, :]

    P, Q, Wec, bec = _node_pre(h, Wr, Wc, W_e2, W_c1, be2, bc1)

    xp = jnp.pad(x, ((0, 0), (0, 5)))
    eap = jnp.pad(edge_attr, ((0, 0), (0, 3)))

    G1 = jnp.take(P, row, axis=0)
    G2 = jnp.take(Q, col, axis=0)
    xr = jnp.take(xp, row, axis=0)
    xc = jnp.take(xp, col, axis=0)

    s, fc = _edge_mlp(G1, G2, xr, xc, eap, Ws, Wa, be1, Wec, bec, Wc2, bc2)

    S = jax.ops.segment_sum(s, row, num_segments=N)
    FC = jax.ops.segment_sum(fc, row, num_segments=N)

    hn, xf = _node_post(h, S, FC, xp, W_e2, be2, Wn1h, Wn1t, bn1, W_n2, bn2)
    return (xf[:, :3], hn)


# speedup vs baseline: 1.8030x; 1.8030x over previous
import functools
import jax
import jax.numpy as jnp
from jax import lax
from jax.experimental import pallas as pl
from jax.experimental.pallas import tpu as pltpu
from jax.experimental.pallas import tpu_sc as plsc

N = 10000
E = 320000
D = 128

NCHUNK = E // 125
EP = NCHUNK * 128
NW = 32
CPW = NCHUNK // NW
TW = 144
HALF = 5008
ACC_R = HALF + 16
NA = 2 * HALF

_PREC = lax.Precision.HIGHEST


def _silu(v):
    return v * jax.nn.sigmoid(v)


def _sc_gather(T1, T2, rowc, colc):
    mesh = plsc.VectorSubcoreMesh(core_axis_name="c", subcore_axis_name="s")

    @functools.partial(
        pl.kernel, mesh=mesh,
        compiler_params=pltpu.CompilerParams(use_tc_tiling_on_sc=False),
        out_type=[
            jax.ShapeDtypeStruct((EP, TW), jnp.float32),
            jax.ShapeDtypeStruct((EP, TW), jnp.float32),
        ],
        scratch_types=[
            pltpu.VMEM((1, 128), jnp.int32),
            pltpu.VMEM((1, 128), jnp.int32),
            pltpu.VMEM((128, TW), jnp.float32),
            pltpu.VMEM((128, TW), jnp.float32),
            pltpu.SemaphoreType.DMA,
            pltpu.SemaphoreType.DMA,
        ],
    )
    def k(T1_hbm, T2_hbm, rowc_hbm, colc_hbm, G1_hbm, G2_hbm,
          idxr, idxc, g1, g2, sem1, sem2):
        wid = lax.axis_index("s") * 2 + lax.axis_index("c")
        base = wid * CPW

        @pl.loop(0, CPW)
        def _(j):
            chunk = base + j
            pltpu.sync_copy(rowc_hbm.at[chunk], idxr.at[0])
            pltpu.sync_copy(colc_hbm.at[chunk], idxc.at[0])
            cp1 = pltpu.async_copy(T1_hbm.at[idxr.at[0]], g1, sem1)
            cp2 = pltpu.async_copy(T2_hbm.at[idxc.at[0]], g2, sem2)
            cp1.wait()
            cp2.wait()
            pltpu.sync_copy(g1, G1_hbm.at[pl.ds(chunk * 128, 128)])
            pltpu.sync_copy(g2, G2_hbm.at[pl.ds(chunk * 128, 128)])

    return k(T1, T2, rowc, colc)


def _sc_scatter(sf, rowc_s):
    mesh = plsc.VectorSubcoreMesh(core_axis_name="c", subcore_axis_name="s")

    @functools.partial(
        pl.kernel, mesh=mesh,
        compiler_params=pltpu.CompilerParams(use_tc_tiling_on_sc=False),
        out_type=jax.ShapeDtypeStruct((NA, TW), jnp.float32),
        scratch_types=[
            pltpu.VMEM((1, 128), jnp.int32),
            pltpu.VMEM((1, 128), jnp.int32),
            pltpu.VMEM((128, TW), jnp.float32),
            pltpu.VMEM((157, TW), jnp.float32),
            pltpu.VMEM_SHARED((ACC_R, TW), jnp.float32),
        ],
    )
    def k(sf_hbm, rowc_hbm, out_hbm, idxs, idx2, sfb, zbuf, accum):
        cid = lax.axis_index("c")
        sid = lax.axis_index("s")
        lo = cid * HALF
        base = sid * (NCHUNK // 16)

        @pl.loop(0, 157)
        def _(r):
            @pl.loop(0, TW, step=16)
            def _(v):
                zbuf[r, pl.ds(v, 16)] = jnp.zeros((16,), jnp.float32)

        zlo = sid * 314
        pltpu.sync_copy(zbuf, accum.at[pl.ds(zlo, 157)])
        pltpu.sync_copy(zbuf, accum.at[pl.ds(zlo + 157, 157)])
        plsc.subcore_barrier()

        @pl.loop(0, NCHUNK // 16)
        def _(j):
            chunk = base + j
            pltpu.sync_copy(rowc_hbm.at[chunk], idxs.at[0])
            pltpu.sync_copy(sf_hbm.at[pl.ds(chunk * 128, 128)], sfb)
            for v in range(0, 128, 16):
                w = idxs[0, pl.ds(v, 16)] - lo
                ok = (w >= 0) & (w < HALF)
                idx2[0, pl.ds(v, 16)] = jnp.where(ok, w, HALF)
            pltpu.sync_copy(sfb, accum.at[idx2.at[0]], add=True)

        plsc.subcore_barrier()
        pltpu.sync_copy(accum.at[pl.ds(sid * 313, 313)],
                        out_hbm.at[pl.ds(lo + sid * 313, 313)])

    return k(sf, rowc_s)


def _pre_body(h_ref, xp_ref, Wr_ref, Wc_ref, We2_ref, Wc1_ref, be2_ref, bc1_ref,
              T1_ref, T2_ref, Wec_ref, bec_ref):
    h = h_ref[...]
    xp = xp_ref[...]
    T1_ref[:, :D] = jnp.dot(h, Wr_ref[...], precision=_PREC)
    T1_ref[:, D:TW] = xp
    T2_ref[:, :D] = jnp.dot(h, Wc_ref[...], precision=_PREC)
    T2_ref[:, D:TW] = xp
    Wec_ref[...] = jnp.dot(We2_ref[...], Wc1_ref[...], precision=_PREC)
    bec_ref[...] = jnp.dot(be2_ref[...], Wc1_ref[...], precision=_PREC) + bc1_ref[...]


def _node_pre(h, xp16, Wr, Wc, We2, Wc1, be2, bc1):
    NB = 1000
    grid = (N // NB,)
    return pl.pallas_call(
        _pre_body,
        grid=grid,
        in_specs=[
            pl.BlockSpec((NB, D), lambda i: (i, 0)),
            pl.BlockSpec((NB, 16), lambda i: (i, 0)),
            pl.BlockSpec((D, D), lambda i: (0, 0)),
            pl.BlockSpec((D, D), lambda i: (0, 0)),
            pl.BlockSpec((D, D), lambda i: (0, 0)),
            pl.BlockSpec((D, D), lambda i: (0, 0)),
            pl.BlockSpec((1, D), lambda i: (0, 0)),
            pl.BlockSpec((1, D), lambda i: (0, 0)),
        ],
        out_specs=[
            pl.BlockSpec((NB, TW), lambda i: (i, 0)),
            pl.BlockSpec((NB, TW), lambda i: (i, 0)),
            pl.BlockSpec((D, D), lambda i: (0, 0)),
            pl.BlockSpec((1, D), lambda i: (0, 0)),
        ],
        out_shape=[
            jax.ShapeDtypeStruct((N, TW), jnp.float32),
            jax.ShapeDtypeStruct((N, TW), jnp.float32),
            jax.ShapeDtypeStruct((D, D), jnp.float32),
            jax.ShapeDtypeStruct((1, D), jnp.float32),
        ],
    )(h, xp16, Wr, Wc, We2, Wc1, be2, bc1)


def _edge_body(G1_ref, G2_ref, ea_ref,
               Ws_ref, Wa_ref, be1_ref, Wec_ref, bec_ref, Wc2_ref, bc2_ref,
               sf_ref):
    G1 = G1_ref[...]
    G2 = G2_ref[...]
    rij = G1[:, D:TW] - G2[:, D:TW]
    scalar = jnp.sum(rij * rij, axis=1, keepdims=True)
    u = (G1[:, :D] + G2[:, :D]
         + scalar * Ws_ref[...]
         + jnp.dot(ea_ref[...], Wa_ref[...], precision=_PREC)
         + be1_ref[...])
    s = _silu(u)
    sf_ref[:, :D] = s
    t = jnp.dot(s, Wec_ref[...], precision=_PREC) + bec_ref[...]
    cm = jnp.dot(_silu(t), Wc2_ref[...], precision=_PREC) + bc2_ref[...]
    ones = (lax.broadcasted_iota(jnp.int32, rij.shape, 1) == 3).astype(jnp.float32)
    sf_ref[:, D:TW] = rij * cm[:, 0:1] + ones


def _edge_mlp(G1, G2, ea, Ws, Wa, be1, Wec, bec, Wc2, bc2):
    EB = 4096
    grid = (EP // EB,)
    return pl.pallas_call(
        _edge_body,
        grid=grid,
        in_specs=[
            pl.BlockSpec((EB, TW), lambda i: (i, 0)),
            pl.BlockSpec((EB, TW), lambda i: (i, 0)),
            pl.BlockSpec((EB, 16), lambda i: (i, 0)),
            pl.BlockSpec((1, D), lambda i: (0, 0)),
            pl.BlockSpec((16, D), lambda i: (0, 0)),
            pl.BlockSpec((1, D), lambda i: (0, 0)),
            pl.BlockSpec((D, D), lambda i: (0, 0)),
            pl.BlockSpec((1, D), lambda i: (0, 0)),
            pl.BlockSpec((D, 128), lambda i: (0, 0)),
            pl.BlockSpec((1, 128), lambda i: (0, 0)),
        ],
        out_specs=[
            pl.BlockSpec((EB, TW), lambda i: (i, 0)),
        ],
        out_shape=[
            jax.ShapeDtypeStruct((EP, TW), jnp.float32),
        ],
    )(G1, G2, ea, Ws, Wa, be1, Wec, bec, Wc2, bc2)


def _post_body(h_ref, SF_ref, xp_ref,
               We2_ref, be2_ref, Wn1h_ref, Wn1t_ref, bn1_ref, Wn2_ref, bn2_ref,
               hn_ref, xf_ref):
    h = h_ref[...]
    SF = SF_ref[...]
    S = SF[:, :D]
    FC = SF[:, D:D + 16]
    cnt = FC[:, 3:4]
    tot = jnp.dot(S, We2_ref[...], precision=_PREC) + cnt * be2_ref[...]
    a = (jnp.dot(h, Wn1h_ref[...], precision=_PREC)
         + jnp.dot(tot, Wn1t_ref[...], precision=_PREC) + bn1_ref[...])
    hn_ref[...] = h + jnp.dot(_silu(a), Wn2_ref[...], precision=_PREC) + bn2_ref[...]
    tot_f = jnp.clip(FC / jnp.maximum(cnt, 1.0), -100.0, 100.0)
    xf_ref[...] = xp_ref[...] + tot_f


def _node_post(h, SF, xp16, We2, be2, Wn1h, Wn1t, bn1, Wn2, bn2):
    NB = 1000
    grid = (N // NB,)
    return pl.pallas_call(
        _post_body,
        grid=grid,
        in_specs=[
            pl.BlockSpec((NB, D), lambda i: (i, 0)),
            pl.BlockSpec((NB, TW), lambda i: (i, 0)),
            pl.BlockSpec((NB, 16), lambda i: (i, 0)),
            pl.BlockSpec((D, D), lambda i: (0, 0)),
            pl.BlockSpec((1, D), lambda i: (0, 0)),
            pl.BlockSpec((D, D), lambda i: (0, 0)),
            pl.BlockSpec((D, D), lambda i: (0, 0)),
            pl.BlockSpec((1, D), lambda i: (0, 0)),
            pl.BlockSpec((D, D), lambda i: (0, 0)),
            pl.BlockSpec((1, D), lambda i: (0, 0)),
        ],
        out_specs=[
            pl.BlockSpec((NB, D), lambda i: (i, 0)),
            pl.BlockSpec((NB, 16), lambda i: (i, 0)),
        ],
        out_shape=[
            jax.ShapeDtypeStruct((N, D), jnp.float32),
            jax.ShapeDtypeStruct((N, 16), jnp.float32),
        ],
    )(h, SF, xp16, We2, be2, Wn1h, Wn1t, bn1, Wn2, bn2)


@jax.jit
def kernel(x, h, edge_index, edge_attr,
           W_e1, b_e1, W_e2, b_e2,
           W_c1, b_c1, W_c2, b_c2,
           W_n1, b_n1, W_n2, b_n2):
    row = edge_index[0].astype(jnp.int32)
    col = edge_index[1].astype(jnp.int32)
    Ws = W_e1[0:1]
    Wr = W_e1[1:1 + D]
    Wc = W_e1[1 + D:1 + 2 * D]
    Wa = jnp.zeros((16, D), jnp.float32).at[:5].set(W_e1[1 + 2 * D:])
    be1 = b_e1[None, :]
    be2 = b_e2[None, :]
    bc1 = b_c1[None, :]
    bc2 = jnp.zeros((1, 128), jnp.float32).at[0, 0].set(b_c2[0])
    Wc2 = jnp.zeros((D, 128), jnp.float32).at[:, 0:1].set(W_c2)
    Wn1h = W_n1[:D]
    Wn1t = W_n1[D:]
    bn1 = b_n1[None, :]
    bn2 = b_n2[None, :]

    xp16 = jnp.pad(x, ((0, 0), (0, 13)))
    T1, T2, Wec, bec = _node_pre(h, xp16, Wr, Wc, W_e2, W_c1, be2, bc1)

    rowc_g = jnp.pad(row.reshape(NCHUNK, 125), ((0, 0), (0, 3)))
    colc_g = jnp.pad(col.reshape(NCHUNK, 125), ((0, 0), (0, 3)))
    rowc_s = jnp.pad(row.reshape(NCHUNK, 125), ((0, 0), (0, 3)),
                     constant_values=N)
    ea16 = jnp.pad(edge_attr.reshape(NCHUNK, 125, 5),
                   ((0, 0), (0, 3), (0, 11))).reshape(EP, 16)

    G1, G2 = _sc_gather(T1, T2, rowc_g, colc_g)

    sf, = _edge_mlp(G1, G2, ea16, Ws, Wa, be1, Wec, bec, Wc2, bc2)

    SF = _sc_scatter(sf, rowc_s)

    hn, xf = _node_post(h, SF[:N], xp16, W_e2, be2, Wn1h, Wn1t, bn1, W_n2, bn2)
    return (xf[:, :3], hn)


# speedup vs baseline: 2.4136x; 1.3387x over previous
import functools
import jax
import jax.numpy as jnp
from jax import lax
from jax.experimental import pallas as pl
from jax.experimental.pallas import tpu as pltpu
from jax.experimental.pallas import tpu_sc as plsc

N = 10000
E = 320000
D = 128

NCHUNK = E // 125
EP = NCHUNK * 128
NW = 32
CPW = NCHUNK // NW
TW = 144
HALF = 5008
ACC_R = HALF + 16
NA = 2 * HALF

_PREC = lax.Precision.HIGHEST


def _silu(v):
    return v * jax.nn.sigmoid(v)


def _sc_gather(T1, T2, X16, rowc, colc):
    mesh = plsc.VectorSubcoreMesh(core_axis_name="c", subcore_axis_name="s")

    @functools.partial(
        pl.kernel, mesh=mesh,
        compiler_params=pltpu.CompilerParams(use_tc_tiling_on_sc=False),
        out_type=[
            jax.ShapeDtypeStruct((EP, D), jnp.float32),
            jax.ShapeDtypeStruct((EP, D), jnp.float32),
            jax.ShapeDtypeStruct((EP, 16), jnp.float32),
            jax.ShapeDtypeStruct((EP, 16), jnp.float32),
        ],
        scratch_types=[
            pltpu.VMEM((CPW, 128), jnp.int32),
            pltpu.VMEM((CPW, 128), jnp.int32),
            pltpu.VMEM((2, 128, D), jnp.float32),
            pltpu.VMEM((2, 128, D), jnp.float32),
            pltpu.VMEM((2, 128, 16), jnp.float32),
            pltpu.VMEM((2, 128, 16), jnp.float32),
            pltpu.SemaphoreType.DMA((2,)),
            pltpu.SemaphoreType.DMA((2,)),
        ],
    )
    def k(T1_hbm, T2_hbm, X16_hbm, rowc_hbm, colc_hbm,
          G1_hbm, G2_hbm, XR_hbm, XC_hbm,
          idxr, idxc, g1, g2, xr, xc, gsem, wsem):
        wid = lax.axis_index("s") * 2 + lax.axis_index("c")
        base = wid * CPW
        pltpu.sync_copy(rowc_hbm.at[pl.ds(base, CPW)], idxr)
        pltpu.sync_copy(colc_hbm.at[pl.ds(base, CPW)], idxc)

        def fire_gather(j, b):
            pltpu.async_copy(T1_hbm.at[idxr.at[j]], g1.at[b], gsem.at[b])
            pltpu.async_copy(T2_hbm.at[idxc.at[j]], g2.at[b], gsem.at[b])
            pltpu.async_copy(X16_hbm.at[idxr.at[j]], xr.at[b], gsem.at[b])
            pltpu.async_copy(X16_hbm.at[idxc.at[j]], xc.at[b], gsem.at[b])

        def wait_gather(j, b):
            pltpu.make_async_copy(T1_hbm.at[idxr.at[j]], g1.at[b], gsem.at[b]).wait()
            pltpu.make_async_copy(T2_hbm.at[idxc.at[j]], g2.at[b], gsem.at[b]).wait()
            pltpu.make_async_copy(X16_hbm.at[idxr.at[j]], xr.at[b], gsem.at[b]).wait()
            pltpu.make_async_copy(X16_hbm.at[idxc.at[j]], xc.at[b], gsem.at[b]).wait()

        def out_slot(chunk):
            return pl.ds(chunk * 128, 128)

        def fire_write(j, b):
            chunk = base + j
            pltpu.async_copy(g1.at[b], G1_hbm.at[out_slot(chunk)], wsem.at[b])
            pltpu.async_copy(g2.at[b], G2_hbm.at[out_slot(chunk)], wsem.at[b])
            pltpu.async_copy(xr.at[b], XR_hbm.at[out_slot(chunk)], wsem.at[b])
            pltpu.async_copy(xc.at[b], XC_hbm.at[out_slot(chunk)], wsem.at[b])

        def wait_write(j, b):
            chunk = base + j
            pltpu.make_async_copy(g1.at[b], G1_hbm.at[out_slot(chunk)], wsem.at[b]).wait()
            pltpu.make_async_copy(g2.at[b], G2_hbm.at[out_slot(chunk)], wsem.at[b]).wait()
            pltpu.make_async_copy(xr.at[b], XR_hbm.at[out_slot(chunk)], wsem.at[b]).wait()
            pltpu.make_async_copy(xc.at[b], XC_hbm.at[out_slot(chunk)], wsem.at[b]).wait()

        fire_gather(0, 0)

        @pl.loop(0, CPW, step=2)
        def _(j):
            for b in (0, 1):
                jj = j + b

                @pl.when(jj >= 1)
                def _():
                    wait_write(jj - 1, 1 - b)

                @pl.when(jj + 1 < CPW)
                def _():
                    fire_gather(jj + 1, 1 - b)

                wait_gather(jj, b)
                fire_write(jj, b)

        wait_write(CPW - 1, 1)

    return k(T1, T2, X16, rowc, colc)


def _sc_scatter(s, fc, rowc_s):
    mesh = plsc.VectorSubcoreMesh(core_axis_name="c", subcore_axis_name="s")
    CPS = NCHUNK // 16

    @functools.partial(
        pl.kernel, mesh=mesh,
        compiler_params=pltpu.CompilerParams(use_tc_tiling_on_sc=False),
        out_type=[
            jax.ShapeDtypeStruct((NA, D), jnp.float32),
            jax.ShapeDtypeStruct((NA, 16), jnp.float32),
        ],
        scratch_types=[
            pltpu.VMEM((CPS, 128), jnp.int32),
            pltpu.VMEM((2, 128), jnp.int32),
            pltpu.VMEM((2, 128, D), jnp.float32),
            pltpu.VMEM((2, 128, 16), jnp.float32),
            pltpu.VMEM((157, D), jnp.float32),
            pltpu.VMEM((157, 16), jnp.float32),
            pltpu.VMEM_SHARED((ACC_R, D), jnp.float32),
            pltpu.VMEM_SHARED((ACC_R, 16), jnp.float32),
            pltpu.SemaphoreType.DMA((2,)),
            pltpu.SemaphoreType.DMA((2,)),
        ],
    )
    def k(s_hbm, fc_hbm, rowc_hbm, sout_hbm, fcout_hbm,
          idxs, idx2, sb, fb, zs, zf, accS, accF, lsem, ssem):
        cid = lax.axis_index("c")
        sid = lax.axis_index("s")
        lo = cid * HALF
        base = sid * CPS

        @pl.loop(0, 157)
        def _(r):
            @pl.loop(0, D, step=16)
            def _(v):
                zs[r, pl.ds(v, 16)] = jnp.zeros((16,), jnp.float32)
            zf[r, pl.ds(0, 16)] = jnp.zeros((16,), jnp.float32)

        zlo = sid * 314
        pltpu.sync_copy(zs, accS.at[pl.ds(zlo, 157)])
        pltpu.sync_copy(zs, accS.at[pl.ds(zlo + 157, 157)])
        pltpu.sync_copy(zf, accF.at[pl.ds(zlo, 157)])
        pltpu.sync_copy(zf, accF.at[pl.ds(zlo + 157, 157)])
        pltpu.sync_copy(rowc_hbm.at[pl.ds(base, CPS)], idxs)
        plsc.subcore_barrier()

        def in_slot(j):
            return pl.ds((base + j) * 128, 128)

        def fire_load(j, b):
            pltpu.async_copy(s_hbm.at[in_slot(j)], sb.at[b], lsem.at[b])
            pltpu.async_copy(fc_hbm.at[in_slot(j)], fb.at[b], lsem.at[b])

        def wait_load(j, b):
            pltpu.make_async_copy(s_hbm.at[in_slot(j)], sb.at[b], lsem.at[b]).wait()
            pltpu.make_async_copy(fc_hbm.at[in_slot(j)], fb.at[b], lsem.at[b]).wait()

        def fire_scatter(j, b):
            pltpu.async_copy(sb.at[b], accS.at[idx2.at[b]], ssem.at[b], add=True)
            pltpu.async_copy(fb.at[b], accF.at[idx2.at[b]], ssem.at[b], add=True)

        def wait_scatter(j, b):
            pltpu.make_async_copy(sb.at[b], accS.at[idx2.at[b]], ssem.at[b]).wait()
            pltpu.make_async_copy(fb.at[b], accF.at[idx2.at[b]], ssem.at[b]).wait()

        fire_load(0, 0)

        @pl.loop(0, CPS, step=2)
        def _(j):
            for b in (0, 1):
                jj = j + b

                @pl.when(jj >= 1)
                def _():
                    wait_scatter(jj - 1, 1 - b)

                @pl.when(jj + 1 < CPS)
                def _():
                    fire_load(jj + 1, 1 - b)

                wait_load(jj, b)
                for v in range(0, 128, 16):
                    w = idxs[jj, pl.ds(v, 16)] - lo
                    ok = (w >= 0) & (w < HALF)
                    idx2[b, pl.ds(v, 16)] = jnp.where(ok, w, HALF)
                fire_scatter(jj, b)

        wait_scatter(CPS - 1, 1)
        plsc.subcore_barrier()
        pltpu.sync_copy(accS.at[pl.ds(sid * 313, 313)],
                        sout_hbm.at[pl.ds(lo + sid * 313, 313)])
        pltpu.sync_copy(accF.at[pl.ds(sid * 313, 313)],
                        fcout_hbm.at[pl.ds(lo + sid * 313, 313)])

    return k(s, fc, rowc_s)


def _pre_body(h_ref, Wr_ref, Wc_ref, We2_ref, Wc1_ref, be2_ref, bc1_ref,
              T1_ref, T2_ref, Wec_ref, bec_ref):
    h = h_ref[...]
    T1_ref[...] = jnp.dot(h, Wr_ref[...], precision=_PREC)
    T2_ref[...] = jnp.dot(h, Wc_ref[...], precision=_PREC)
    Wec_ref[...] = jnp.dot(We2_ref[...], Wc1_ref[...], precision=_PREC)
    bec_ref[...] = jnp.dot(be2_ref[...], Wc1_ref[...], precision=_PREC) + bc1_ref[...]


def _node_pre(h, Wr, Wc, We2, Wc1, be2, bc1):
    NB = 1000
    grid = (N // NB,)
    return pl.pallas_call(
        _pre_body,
        grid=grid,
        in_specs=[
            pl.BlockSpec((NB, D), lambda i: (i, 0)),
            pl.BlockSpec((D, D), lambda i: (0, 0)),
            pl.BlockSpec((D, D), lambda i: (0, 0)),
            pl.BlockSpec((D, D), lambda i: (0, 0)),
            pl.BlockSpec((D, D), lambda i: (0, 0)),
            pl.BlockSpec((1, D), lambda i: (0, 0)),
            pl.BlockSpec((1, D), lambda i: (0, 0)),
        ],
        out_specs=[
            pl.BlockSpec((NB, D), lambda i: (i, 0)),
            pl.BlockSpec((NB, D), lambda i: (i, 0)),
            pl.BlockSpec((D, D), lambda i: (0, 0)),
            pl.BlockSpec((1, D), lambda i: (0, 0)),
        ],
        out_shape=[
            jax.ShapeDtypeStruct((N, D), jnp.float32),
            jax.ShapeDtypeStruct((N, D), jnp.float32),
            jax.ShapeDtypeStruct((D, D), jnp.float32),
            jax.ShapeDtypeStruct((1, D), jnp.float32),
        ],
    )(h, Wr, Wc, We2, Wc1, be2, bc1)


def _edge_body(G1_ref, G2_ref, XR_ref, XC_ref, ea_ref,
               Ws_ref, Wa_ref, be1_ref, Wec_ref, bec_ref, Wc2_ref, bc2_ref,
               s_ref, fc_ref):
    rij = XR_ref[...] - XC_ref[...]
    scalar = jnp.sum(rij * rij, axis=1, keepdims=True)
    u = (G1_ref[...] + G2_ref[...]
         + scalar * Ws_ref[...]
         + jnp.dot(ea_ref[...], Wa_ref[...], precision=_PREC)
         + be1_ref[...])
    s = _silu(u)
    s_ref[...] = s
    t = jnp.dot(s, Wec_ref[...], precision=_PREC) + bec_ref[...]
    cm = jnp.dot(_silu(t), Wc2_ref[...], precision=_PREC) + bc2_ref[...]
    ones = (lax.broadcasted_iota(jnp.int32, rij.shape, 1) == 3).astype(jnp.float32)
    fc_ref[...] = rij * cm[:, 0:1] + ones


def _edge_mlp(G1, G2, XR, XC, ea, Ws, Wa, be1, Wec, bec, Wc2, bc2):
    EB = 4096
    grid = (EP // EB,)
    return pl.pallas_call(
        _edge_body,
        grid=grid,
        in_specs=[
            pl.BlockSpec((EB, D), lambda i: (i, 0)),
            pl.BlockSpec((EB, D), lambda i: (i, 0)),
            pl.BlockSpec((EB, 16), lambda i: (i, 0)),
            pl.BlockSpec((EB, 16), lambda i: (i, 0)),
            pl.BlockSpec((EB, 16), lambda i: (i, 0)),
            pl.BlockSpec((1, D), lambda i: (0, 0)),
            pl.BlockSpec((16, D), lambda i: (0, 0)),
            pl.BlockSpec((1, D), lambda i: (0, 0)),
            pl.BlockSpec((D, D), lambda i: (0, 0)),
            pl.BlockSpec((1, D), lambda i: (0, 0)),
            pl.BlockSpec((D, 128), lambda i: (0, 0)),
            pl.BlockSpec((1, 128), lambda i: (0, 0)),
        ],
        out_specs=[
            pl.BlockSpec((EB, D), lambda i: (i, 0)),
            pl.BlockSpec((EB, 16), lambda i: (i, 0)),
        ],
        out_shape=[
            jax.ShapeDtypeStruct((EP, D), jnp.float32),
            jax.ShapeDtypeStruct((EP, 16), jnp.float32),
        ],
    )(G1, G2, XR, XC, ea, Ws, Wa, be1, Wec, bec, Wc2, bc2)


def _post_body(h_ref, S_ref, FC_ref, xp_ref,
               We2_ref, be2_ref, Wn1h_ref, Wn1t_ref, bn1_ref, Wn2_ref, bn2_ref,
               hn_ref, xf_ref):
    h = h_ref[...]
    S = S_ref[...]
    FC = FC_ref[...]
    cnt = FC[:, 3:4]
    tot = jnp.dot(S, We2_ref[...], precision=_PREC) + cnt * be2_ref[...]
    a = (jnp.dot(h, Wn1h_ref[...], precision=_PREC)
         + jnp.dot(tot, Wn1t_ref[...], precision=_PREC) + bn1_ref[...])
    hn_ref[...] = h + jnp.dot(_silu(a), Wn2_ref[...], precision=_PREC) + bn2_ref[...]
    tot_f = jnp.clip(FC / jnp.maximum(cnt, 1.0), -100.0, 100.0)
    xf_ref[...] = xp_ref[...] + tot_f


def _node_post(h, S, FC, xp16, We2, be2, Wn1h, Wn1t, bn1, Wn2, bn2):
    NB = 1000
    grid = (N // NB,)
    return pl.pallas_call(
        _post_body,
        grid=grid,
        in_specs=[
            pl.BlockSpec((NB, D), lambda i: (i, 0)),
            pl.BlockSpec((NB, D), lambda i: (i, 0)),
            pl.BlockSpec((NB, 16), lambda i: (i, 0)),
            pl.BlockSpec((NB, 16), lambda i: (i, 0)),
            pl.BlockSpec((D, D), lambda i: (0, 0)),
            pl.BlockSpec((1, D), lambda i: (0, 0)),
            pl.BlockSpec((D, D), lambda i: (0, 0)),
            pl.BlockSpec((D, D), lambda i: (0, 0)),
            pl.BlockSpec((1, D), lambda i: (0, 0)),
            pl.BlockSpec((D, D), lambda i: (0, 0)),
            pl.BlockSpec((1, D), lambda i: (0, 0)),
        ],
        out_specs=[
            pl.BlockSpec((NB, D), lambda i: (i, 0)),
            pl.BlockSpec((NB, 16), lambda i: (i, 0)),
        ],
        out_shape=[
            jax.ShapeDtypeStruct((N, D), jnp.float32),
            jax.ShapeDtypeStruct((N, 16), jnp.float32),
        ],
    )(h, S, FC, xp16, We2, be2, Wn1h, Wn1t, bn1, Wn2, bn2)


@jax.jit
def kernel(x, h, edge_index, edge_attr,
           W_e1, b_e1, W_e2, b_e2,
           W_c1, b_c1, W_c2, b_c2,
           W_n1, b_n1, W_n2, b_n2):
    row = edge_index[0].astype(jnp.int32)
    col = edge_index[1].astype(jnp.int32)
    Ws = W_e1[0:1]
    Wr = W_e1[1:1 + D]
    Wc = W_e1[1 + D:1 + 2 * D]
    Wa = jnp.zeros((16, D), jnp.float32).at[:5].set(W_e1[1 + 2 * D:])
    be1 = b_e1[None, :]
    be2 = b_e2[None, :]
    bc1 = b_c1[None, :]
    bc2 = jnp.zeros((1, 128), jnp.float32).at[0, 0].set(b_c2[0])
    Wc2 = jnp.zeros((D, 128), jnp.float32).at[:, 0:1].set(W_c2)
    Wn1h = W_n1[:D]
    Wn1t = W_n1[D:]
    bn1 = b_n1[None, :]
    bn2 = b_n2[None, :]

    xp16 = jnp.pad(x, ((0, 0), (0, 13)))
    T1, T2, Wec, bec = _node_pre(h, Wr, Wc, W_e2, W_c1, be2, bc1)

    rowc_g = jnp.pad(row.reshape(NCHUNK, 125), ((0, 0), (0, 3)))
    colc_g = jnp.pad(col.reshape(NCHUNK, 125), ((0, 0), (0, 3)))
    rowc_s = jnp.pad(row.reshape(NCHUNK, 125), ((0, 0), (0, 3)),
                     constant_values=N)
    ea16 = jnp.pad(edge_attr.reshape(NCHUNK, 125, 5),
                   ((0, 0), (0, 3), (0, 11))).reshape(EP, 16)

    G1, G2, XR, XC = _sc_gather(T1, T2, xp16, rowc_g, colc_g)

    s, fc = _edge_mlp(G1, G2, XR, XC, ea16, Ws, Wa, be1, Wec, bec, Wc2, bc2)

    SOUT, FCOUT = _sc_scatter(s, fc, rowc_s)

    hn, xf = _node_post(h, SOUT[:N], FCOUT[:N], xp16, W_e2, be2,
                        Wn1h, Wn1t, bn1, W_n2, bn2)
    return (xf[:, :3], hn)


# speedup vs baseline: 2.5996x; 1.0770x over previous
import functools
import jax
import jax.numpy as jnp
from jax import lax
from jax.experimental import pallas as pl
from jax.experimental.pallas import tpu as pltpu
from jax.experimental.pallas import tpu_sc as plsc

N = 10000
E = 320000
D = 128

NCHUNK = E // 125
EP = NCHUNK * 128
NW = 32
CPW = NCHUNK // NW
TW = 144
HALF = 5008
ACC_R = HALF + 16
NA = 2 * HALF

_PREC = lax.Precision.HIGHEST


def _silu(v):
    return v * jax.nn.sigmoid(v)


def _sc_gather(T1, T2, X16, rowc, colc, nchunk):
    mesh = plsc.VectorSubcoreMesh(core_axis_name="c", subcore_axis_name="s")
    CPW = nchunk // NW
    EPH = nchunk * 128

    @functools.partial(
        pl.kernel, mesh=mesh,
        compiler_params=pltpu.CompilerParams(use_tc_tiling_on_sc=False),
        out_type=[
            jax.ShapeDtypeStruct((EPH, D), jnp.float32),
            jax.ShapeDtypeStruct((EPH, D), jnp.float32),
            jax.ShapeDtypeStruct((EPH, 16), jnp.float32),
            jax.ShapeDtypeStruct((EPH, 16), jnp.float32),
        ],
        scratch_types=[
            pltpu.VMEM((CPW, 128), jnp.int32),
            pltpu.VMEM((CPW, 128), jnp.int32),
            pltpu.VMEM((2, 128, D), jnp.float32),
            pltpu.VMEM((2, 128, D), jnp.float32),
            pltpu.VMEM((2, 128, 16), jnp.float32),
            pltpu.VMEM((2, 128, 16), jnp.float32),
            pltpu.SemaphoreType.DMA((2,)),
            pltpu.SemaphoreType.DMA((2,)),
        ],
    )
    def k(T1_hbm, T2_hbm, X16_hbm, rowc_hbm, colc_hbm,
          G1_hbm, G2_hbm, XR_hbm, XC_hbm,
          idxr, idxc, g1, g2, xr, xc, gsem, wsem):
        wid = lax.axis_index("s") * 2 + lax.axis_index("c")
        base = wid * CPW
        pltpu.sync_copy(rowc_hbm.at[pl.ds(base, CPW)], idxr)
        pltpu.sync_copy(colc_hbm.at[pl.ds(base, CPW)], idxc)

        def fire_gather(j, b):
            pltpu.async_copy(T1_hbm.at[idxr.at[j]], g1.at[b], gsem.at[b])
            pltpu.async_copy(T2_hbm.at[idxc.at[j]], g2.at[b], gsem.at[b])
            pltpu.async_copy(X16_hbm.at[idxr.at[j]], xr.at[b], gsem.at[b])
            pltpu.async_copy(X16_hbm.at[idxc.at[j]], xc.at[b], gsem.at[b])

        def wait_gather(j, b):
            pltpu.make_async_copy(T1_hbm.at[idxr.at[j]], g1.at[b], gsem.at[b]).wait()
            pltpu.make_async_copy(T2_hbm.at[idxc.at[j]], g2.at[b], gsem.at[b]).wait()
            pltpu.make_async_copy(X16_hbm.at[idxr.at[j]], xr.at[b], gsem.at[b]).wait()
            pltpu.make_async_copy(X16_hbm.at[idxc.at[j]], xc.at[b], gsem.at[b]).wait()

        def out_slot(chunk):
            return pl.ds(chunk * 128, 128)

        def fire_write(j, b):
            chunk = base + j
            pltpu.async_copy(g1.at[b], G1_hbm.at[out_slot(chunk)], wsem.at[b])
            pltpu.async_copy(g2.at[b], G2_hbm.at[out_slot(chunk)], wsem.at[b])
            pltpu.async_copy(xr.at[b], XR_hbm.at[out_slot(chunk)], wsem.at[b])
            pltpu.async_copy(xc.at[b], XC_hbm.at[out_slot(chunk)], wsem.at[b])

        def wait_write(j, b):
            chunk = base + j
            pltpu.make_async_copy(g1.at[b], G1_hbm.at[out_slot(chunk)], wsem.at[b]).wait()
            pltpu.make_async_copy(g2.at[b], G2_hbm.at[out_slot(chunk)], wsem.at[b]).wait()
            pltpu.make_async_copy(xr.at[b], XR_hbm.at[out_slot(chunk)], wsem.at[b]).wait()
            pltpu.make_async_copy(xc.at[b], XC_hbm.at[out_slot(chunk)], wsem.at[b]).wait()

        fire_gather(0, 0)

        @pl.loop(0, CPW, step=2)
        def _(j):
            for b in (0, 1):
                jj = j + b

                @pl.when(jj >= 1)
                def _():
                    wait_write(jj - 1, 1 - b)

                @pl.when(jj + 1 < CPW)
                def _():
                    fire_gather(jj + 1, 1 - b)

                wait_gather(jj, b)
                fire_write(jj, b)

        wait_write(CPW - 1, 1)

    return k(T1, T2, X16, rowc, colc)


def _sc_scatter(s, fc, rowc_s, nchunk):
    mesh = plsc.VectorSubcoreMesh(core_axis_name="c", subcore_axis_name="s")
    CPS = nchunk // 16

    @functools.partial(
        pl.kernel, mesh=mesh,
        compiler_params=pltpu.CompilerParams(use_tc_tiling_on_sc=False),
        out_type=[
            jax.ShapeDtypeStruct((NA, D), jnp.float32),
            jax.ShapeDtypeStruct((NA, 16), jnp.float32),
        ],
        scratch_types=[
            pltpu.VMEM((CPS, 128), jnp.int32),
            pltpu.VMEM((2, 128), jnp.int32),
            pltpu.VMEM((2, 128, D), jnp.float32),
            pltpu.VMEM((2, 128, 16), jnp.float32),
            pltpu.VMEM((157, D), jnp.float32),
            pltpu.VMEM((157, 16), jnp.float32),
            pltpu.VMEM_SHARED((ACC_R, D), jnp.float32),
            pltpu.VMEM_SHARED((ACC_R, 16), jnp.float32),
            pltpu.SemaphoreType.DMA((2,)),
            pltpu.SemaphoreType.DMA((2,)),
        ],
    )
    def k(s_hbm, fc_hbm, rowc_hbm, sout_hbm, fcout_hbm,
          idxs, idx2, sb, fb, zs, zf, accS, accF, lsem, ssem):
        cid = lax.axis_index("c")
        sid = lax.axis_index("s")
        lo = cid * HALF
        base = sid * CPS

        @pl.loop(0, 157)
        def _(r):
            @pl.loop(0, D, step=16)
            def _(v):
                zs[r, pl.ds(v, 16)] = jnp.zeros((16,), jnp.float32)
            zf[r, pl.ds(0, 16)] = jnp.zeros((16,), jnp.float32)

        zlo = sid * 314
        pltpu.sync_copy(zs, accS.at[pl.ds(zlo, 157)])
        pltpu.sync_copy(zs, accS.at[pl.ds(zlo + 157, 157)])
        pltpu.sync_copy(zf, accF.at[pl.ds(zlo, 157)])
        pltpu.sync_copy(zf, accF.at[pl.ds(zlo + 157, 157)])
        pltpu.sync_copy(rowc_hbm.at[pl.ds(base, CPS)], idxs)
        plsc.subcore_barrier()

        def in_slot(j):
            return pl.ds((base + j) * 128, 128)

        def fire_load(j, b):
            pltpu.async_copy(s_hbm.at[in_slot(j)], sb.at[b], lsem.at[b])
            pltpu.async_copy(fc_hbm.at[in_slot(j)], fb.at[b], lsem.at[b])

        def wait_load(j, b):
            pltpu.make_async_copy(s_hbm.at[in_slot(j)], sb.at[b], lsem.at[b]).wait()
            pltpu.make_async_copy(fc_hbm.at[in_slot(j)], fb.at[b], lsem.at[b]).wait()

        def fire_scatter(j, b):
            pltpu.async_copy(sb.at[b], accS.at[idx2.at[b]], ssem.at[b], add=True)
            pltpu.async_copy(fb.at[b], accF.at[idx2.at[b]], ssem.at[b], add=True)

        def wait_scatter(j, b):
            pltpu.make_async_copy(sb.at[b], accS.at[idx2.at[b]], ssem.at[b]).wait()
            pltpu.make_async_copy(fb.at[b], accF.at[idx2.at[b]], ssem.at[b]).wait()

        fire_load(0, 0)

        @pl.loop(0, CPS, step=2)
        def _(j):
            for b in (0, 1):
                jj = j + b

                @pl.when(jj >= 1)
                def _():
                    wait_scatter(jj - 1, 1 - b)

                @pl.when(jj + 1 < CPS)
                def _():
                    fire_load(jj + 1, 1 - b)

                wait_load(jj, b)
                for v in range(0, 128, 16):
                    w = idxs[jj, pl.ds(v, 16)] - lo
                    ok = (w >= 0) & (w < HALF)
                    idx2[b, pl.ds(v, 16)] = jnp.where(ok, w, HALF)
                fire_scatter(jj, b)

        wait_scatter(CPS - 1, 1)
        plsc.subcore_barrier()
        pltpu.sync_copy(accS.at[pl.ds(sid * 313, 313)],
                        sout_hbm.at[pl.ds(lo + sid * 313, 313)])
        pltpu.sync_copy(accF.at[pl.ds(sid * 313, 313)],
                        fcout_hbm.at[pl.ds(lo + sid * 313, 313)])

    return k(s, fc, rowc_s)


def _pre_body(h_ref, Wr_ref, Wc_ref, We2_ref, Wc1_ref, be2_ref, bc1_ref,
              T1_ref, T2_ref, Wec_ref, bec_ref):
    h = h_ref[...]
    T1_ref[...] = jnp.dot(h, Wr_ref[...], precision=_PREC)
    T2_ref[...] = jnp.dot(h, Wc_ref[...], precision=_PREC)
    Wec_ref[...] = jnp.dot(We2_ref[...], Wc1_ref[...], precision=_PREC)
    bec_ref[...] = jnp.dot(be2_ref[...], Wc1_ref[...], precision=_PREC) + bc1_ref[...]


def _node_pre(h, Wr, Wc, We2, Wc1, be2, bc1):
    NB = 1000
    grid = (N // NB,)
    return pl.pallas_call(
        _pre_body,
        grid=grid,
        in_specs=[
            pl.BlockSpec((NB, D), lambda i: (i, 0)),
            pl.BlockSpec((D, D), lambda i: (0, 0)),
            pl.BlockSpec((D, D), lambda i: (0, 0)),
            pl.BlockSpec((D, D), lambda i: (0, 0)),
            pl.BlockSpec((D, D), lambda i: (0, 0)),
            pl.BlockSpec((1, D), lambda i: (0, 0)),
            pl.BlockSpec((1, D), lambda i: (0, 0)),
        ],
        out_specs=[
            pl.BlockSpec((NB, D), lambda i: (i, 0)),
            pl.BlockSpec((NB, D), lambda i: (i, 0)),
            pl.BlockSpec((D, D), lambda i: (0, 0)),
            pl.BlockSpec((1, D), lambda i: (0, 0)),
        ],
        out_shape=[
            jax.ShapeDtypeStruct((N, D), jnp.float32),
            jax.ShapeDtypeStruct((N, D), jnp.float32),
            jax.ShapeDtypeStruct((D, D), jnp.float32),
            jax.ShapeDtypeStruct((1, D), jnp.float32),
        ],
    )(h, Wr, Wc, We2, Wc1, be2, bc1)


def _edge_body(G1_ref, G2_ref, XR_ref, XC_ref, ea_ref,
               Ws_ref, Wa_ref, be1_ref, Wec_ref, bec_ref, Wc2_ref, bc2_ref,
               s_ref, fc_ref):
    rij = XR_ref[...] - XC_ref[...]
    scalar = jnp.sum(rij * rij, axis=1, keepdims=True)
    u = (G1_ref[...] + G2_ref[...]
         + scalar * Ws_ref[...]
         + jnp.dot(ea_ref[...], Wa_ref[...], precision=_PREC)
         + be1_ref[...])
    s = _silu(u)
    s_ref[...] = s
    t = jnp.dot(s, Wec_ref[...], precision=_PREC) + bec_ref[...]
    cm = jnp.dot(_silu(t), Wc2_ref[...], precision=_PREC) + bc2_ref[...]
    ones = (lax.broadcasted_iota(jnp.int32, rij.shape, 1) == 3).astype(jnp.float32)
    fc_ref[...] = rij * cm[:, 0:1] + ones


def _edge_mlp(G1, G2, XR, XC, ea, Ws, Wa, be1, Wec, bec, Wc2, bc2):
    EB = 4096
    EPH = G1.shape[0]
    grid = (EPH // EB,)
    return pl.pallas_call(
        _edge_body,
        grid=grid,
        in_specs=[
            pl.BlockSpec((EB, D), lambda i: (i, 0)),
            pl.BlockSpec((EB, D), lambda i: (i, 0)),
            pl.BlockSpec((EB, 16), lambda i: (i, 0)),
            pl.BlockSpec((EB, 16), lambda i: (i, 0)),
            pl.BlockSpec((EB, 16), lambda i: (i, 0)),
            pl.BlockSpec((1, D), lambda i: (0, 0)),
            pl.BlockSpec((16, D), lambda i: (0, 0)),
            pl.BlockSpec((1, D), lambda i: (0, 0)),
            pl.BlockSpec((D, D), lambda i: (0, 0)),
            pl.BlockSpec((1, D), lambda i: (0, 0)),
            pl.BlockSpec((D, 128), lambda i: (0, 0)),
            pl.BlockSpec((1, 128), lambda i: (0, 0)),
        ],
        out_specs=[
            pl.BlockSpec((EB, D), lambda i: (i, 0)),
            pl.BlockSpec((EB, 16), lambda i: (i, 0)),
        ],
        out_shape=[
            jax.ShapeDtypeStruct((EPH, D), jnp.float32),
            jax.ShapeDtypeStruct((EPH, 16), jnp.float32),
        ],
    )(G1, G2, XR, XC, ea, Ws, Wa, be1, Wec, bec, Wc2, bc2)


def _post_body(h_ref, S0_ref, S1_ref, FC0_ref, FC1_ref, xp_ref,
               We2_ref, be2_ref, Wn1h_ref, Wn1t_ref, bn1_ref, Wn2_ref, bn2_ref,
               hn_ref, xf_ref):
    h = h_ref[...]
    S = S0_ref[...] + S1_ref[...]
    FC = FC0_ref[...] + FC1_ref[...]
    cnt = FC[:, 3:4]
    tot = jnp.dot(S, We2_ref[...], precision=_PREC) + cnt * be2_ref[...]
    a = (jnp.dot(h, Wn1h_ref[...], precision=_PREC)
         + jnp.dot(tot, Wn1t_ref[...], precision=_PREC) + bn1_ref[...])
    hn_ref[...] = h + jnp.dot(_silu(a), Wn2_ref[...], precision=_PREC) + bn2_ref[...]
    tot_f = jnp.clip(FC / jnp.maximum(cnt, 1.0), -100.0, 100.0)
    xf_ref[...] = xp_ref[...] + tot_f


def _node_post(h, S0, S1, FC0, FC1, xp16, We2, be2, Wn1h, Wn1t, bn1, Wn2, bn2):
    NB = 1000
    grid = (N // NB,)
    return pl.pallas_call(
        _post_body,
        grid=grid,
        in_specs=[
            pl.BlockSpec((NB, D), lambda i: (i, 0)),
            pl.BlockSpec((NB, D), lambda i: (i, 0)),
            pl.BlockSpec((NB, D), lambda i: (i, 0)),
            pl.BlockSpec((NB, 16), lambda i: (i, 0)),
            pl.BlockSpec((NB, 16), lambda i: (i, 0)),
            pl.BlockSpec((NB, 16), lambda i: (i, 0)),
            pl.BlockSpec((D, D), lambda i: (0, 0)),
            pl.BlockSpec((1, D), lambda i: (0, 0)),
            pl.BlockSpec((D, D), lambda i: (0, 0)),
            pl.BlockSpec((D, D), lambda i: (0, 0)),
            pl.BlockSpec((1, D), lambda i: (0, 0)),
            pl.BlockSpec((D, D), lambda i: (0, 0)),
            pl.BlockSpec((1, D), lambda i: (0, 0)),
        ],
        out_specs=[
            pl.BlockSpec((NB, D), lambda i: (i, 0)),
            pl.BlockSpec((NB, 16), lambda i: (i, 0)),
        ],
        out_shape=[
            jax.ShapeDtypeStruct((N, D), jnp.float32),
            jax.ShapeDtypeStruct((N, 16), jnp.float32),
        ],
    )(h, S0, S1, FC0, FC1, xp16, We2, be2, Wn1h, Wn1t, bn1, Wn2, bn2)


@jax.jit
def kernel(x, h, edge_index, edge_attr,
           W_e1, b_e1, W_e2, b_e2,
           W_c1, b_c1, W_c2, b_c2,
           W_n1, b_n1, W_n2, b_n2):
    row = edge_index[0].astype(jnp.int32)
    col = edge_index[1].astype(jnp.int32)
    Ws = W_e1[0:1]
    Wr = W_e1[1:1 + D]
    Wc = W_e1[1 + D:1 + 2 * D]
    Wa = jnp.zeros((16, D), jnp.float32).at[:5].set(W_e1[1 + 2 * D:])
    be1 = b_e1[None, :]
    be2 = b_e2[None, :]
    bc1 = b_c1[None, :]
    bc2 = jnp.zeros((1, 128), jnp.float32).at[0, 0].set(b_c2[0])
    Wc2 = jnp.zeros((D, 128), jnp.float32).at[:, 0:1].set(W_c2)
    Wn1h = W_n1[:D]
    Wn1t = W_n1[D:]
    bn1 = b_n1[None, :]
    bn2 = b_n2[None, :]

    xp16 = jnp.pad(x, ((0, 0), (0, 13)))
    T1, T2, Wec, bec = _node_pre(h, Wr, Wc, W_e2, W_c1, be2, bc1)

    rowc_g = jnp.pad(row.reshape(NCHUNK, 125), ((0, 0), (0, 3)))
    colc_g = jnp.pad(col.reshape(NCHUNK, 125), ((0, 0), (0, 3)))
    rowc_s = jnp.pad(row.reshape(NCHUNK, 125), ((0, 0), (0, 3)),
                     constant_values=N)
    ea16 = jnp.pad(edge_attr.reshape(NCHUNK, 125, 5),
                   ((0, 0), (0, 3), (0, 11))).reshape(EP, 16)

    NH = NCHUNK // 2
    EPH = NH * 128
    souts = []
    fcouts = []
    for hf in (0, 1):
        cs = slice(hf * NH, (hf + 1) * NH)
        es = slice(hf * EPH, (hf + 1) * EPH)
        G1, G2, XR, XC = _sc_gather(T1, T2, xp16, rowc_g[cs], colc_g[cs], NH)
        s, fc = _edge_mlp(G1, G2, XR, XC, ea16[es], Ws, Wa, be1,
                          Wec, bec, Wc2, bc2)
        so, fo = _sc_scatter(s, fc, rowc_s[cs], NH)
        souts.append(so)
        fcouts.append(fo)

    hn, xf = _node_post(h, souts[0][:N], souts[1][:N],
                        fcouts[0][:N], fcouts[1][:N], xp16, W_e2, be2,
                        Wn1h, Wn1t, bn1, W_n2, bn2)
    return (xf[:, :3], hn)


# speedup vs baseline: 2.9981x; 1.1533x over previous
import functools
import jax
import jax.numpy as jnp
from jax import lax
from jax.experimental import pallas as pl
from jax.experimental.pallas import tpu as pltpu
from jax.experimental.pallas import tpu_sc as plsc

N = 10000
E = 320000
D = 128

NCHUNK = E // 125
EP = NCHUNK * 128
NW = 32
CPW = NCHUNK // NW
TW = 144
HALF = 5008
ACC_R = HALF + 16
NA = 2 * HALF

_PREC = lax.Precision.HIGHEST


def _silu(v):
    return v * jax.nn.sigmoid(v)


def _sc_gather(T1, T2, X16, rowc, colc, nchunk):
    mesh = plsc.VectorSubcoreMesh(core_axis_name="c", subcore_axis_name="s")
    CPW = nchunk // NW
    EPH = nchunk * 128

    @functools.partial(
        pl.kernel, mesh=mesh,
        compiler_params=pltpu.CompilerParams(use_tc_tiling_on_sc=False),
        out_type=[
            jax.ShapeDtypeStruct((EPH, D), jnp.float32),
            jax.ShapeDtypeStruct((EPH, 16), jnp.float32),
        ],
        scratch_types=[
            pltpu.VMEM((CPW, 128), jnp.int32),
            pltpu.VMEM((CPW, 128), jnp.int32),
            pltpu.VMEM((2, 128, D), jnp.float32),
            pltpu.VMEM((2, 128, D), jnp.float32),
            pltpu.VMEM((2, 128, 16), jnp.float32),
            pltpu.VMEM((2, 128, 16), jnp.float32),
            pltpu.SemaphoreType.DMA((2,)),
            pltpu.SemaphoreType.DMA((2,)),
        ],
    )
    def k(T1_hbm, T2_hbm, X16_hbm, rowc_hbm, colc_hbm,
          G_hbm, RIJ_hbm,
          idxr, idxc, g1, g2, xr, xc, gsem, wsem):
        wid = lax.axis_index("s") * 2 + lax.axis_index("c")
        base = wid * CPW
        pltpu.sync_copy(rowc_hbm.at[pl.ds(base, CPW)], idxr)
        pltpu.sync_copy(colc_hbm.at[pl.ds(base, CPW)], idxc)

        def fire_gather(j, b):
            pltpu.async_copy(T1_hbm.at[idxr.at[j]], g1.at[b], gsem.at[b])
            pltpu.async_copy(T2_hbm.at[idxc.at[j]], g2.at[b], gsem.at[b])
            pltpu.async_copy(X16_hbm.at[idxr.at[j]], xr.at[b], gsem.at[b])
            pltpu.async_copy(X16_hbm.at[idxc.at[j]], xc.at[b], gsem.at[b])

        def wait_gather(j, b):
            pltpu.make_async_copy(T1_hbm.at[idxr.at[j]], g1.at[b], gsem.at[b]).wait()
            pltpu.make_async_copy(T2_hbm.at[idxc.at[j]], g2.at[b], gsem.at[b]).wait()
            pltpu.make_async_copy(X16_hbm.at[idxr.at[j]], xr.at[b], gsem.at[b]).wait()
            pltpu.make_async_copy(X16_hbm.at[idxc.at[j]], xc.at[b], gsem.at[b]).wait()

        def out_slot(chunk):
            return pl.ds(chunk * 128, 128)

        def fire_write(j, b):
            chunk = base + j
            pltpu.async_copy(g1.at[b], G_hbm.at[out_slot(chunk)], wsem.at[b])
            pltpu.async_copy(xr.at[b], RIJ_hbm.at[out_slot(chunk)], wsem.at[b])

        def wait_write(j, b):
            chunk = base + j
            pltpu.make_async_copy(g1.at[b], G_hbm.at[out_slot(chunk)], wsem.at[b]).wait()
            pltpu.make_async_copy(xr.at[b], RIJ_hbm.at[out_slot(chunk)], wsem.at[b]).wait()

        def combine(b):
            @pl.loop(0, 128)
            def _(r):
                for v in range(0, D, 16):
                    g1[b, r, pl.ds(v, 16)] = (g1[b, r, pl.ds(v, 16)]
                                              + g2[b, r, pl.ds(v, 16)])
                xr[b, r, pl.ds(0, 16)] = (xr[b, r, pl.ds(0, 16)]
                                          - xc[b, r, pl.ds(0, 16)])

        fire_gather(0, 0)

        @pl.loop(0, CPW, step=2)
        def _(j):
            for b in (0, 1):
                jj = j + b

                @pl.when(jj >= 1)
                def _():
                    wait_write(jj - 1, 1 - b)

                @pl.when(jj + 1 < CPW)
                def _():
                    fire_gather(jj + 1, 1 - b)

                wait_gather(jj, b)
                combine(b)
                fire_write(jj, b)

        wait_write(CPW - 1, 1)

    return k(T1, T2, X16, rowc, colc)


def _sc_scatter(s, fc, rowc_s, nchunk):
    mesh = plsc.VectorSubcoreMesh(core_axis_name="c", subcore_axis_name="s")
    CPS = nchunk // 16

    @functools.partial(
        pl.kernel, mesh=mesh,
        compiler_params=pltpu.CompilerParams(use_tc_tiling_on_sc=False),
        out_type=[
            jax.ShapeDtypeStruct((NA, D), jnp.float32),
            jax.ShapeDtypeStruct((NA, 16), jnp.float32),
        ],
        scratch_types=[
            pltpu.VMEM((CPS, 128), jnp.int32),
            pltpu.VMEM((2, 128), jnp.int32),
            pltpu.VMEM((2, 128, D), jnp.float32),
            pltpu.VMEM((2, 128, 16), jnp.float32),
            pltpu.VMEM((157, D), jnp.float32),
            pltpu.VMEM((157, 16), jnp.float32),
            pltpu.VMEM_SHARED((ACC_R, D), jnp.float32),
            pltpu.VMEM_SHARED((ACC_R, 16), jnp.float32),
            pltpu.SemaphoreType.DMA((2,)),
            pltpu.SemaphoreType.DMA((2,)),
        ],
    )
    def k(s_hbm, fc_hbm, rowc_hbm, sout_hbm, fcout_hbm,
          idxs, idx2, sb, fb, zs, zf, accS, accF, lsem, ssem):
        cid = lax.axis_index("c")
        sid = lax.axis_index("s")
        lo = cid * HALF
        base = sid * CPS

        @pl.loop(0, 157)
        def _(r):
            @pl.loop(0, D, step=16)
            def _(v):
                zs[r, pl.ds(v, 16)] = jnp.zeros((16,), jnp.float32)
            zf[r, pl.ds(0, 16)] = jnp.zeros((16,), jnp.float32)

        zlo = sid * 314
        pltpu.sync_copy(zs, accS.at[pl.ds(zlo, 157)])
        pltpu.sync_copy(zs, accS.at[pl.ds(zlo + 157, 157)])
        pltpu.sync_copy(zf, accF.at[pl.ds(zlo, 157)])
        pltpu.sync_copy(zf, accF.at[pl.ds(zlo + 157, 157)])
        pltpu.sync_copy(rowc_hbm.at[pl.ds(base, CPS)], idxs)
        plsc.subcore_barrier()

        def in_slot(j):
            return pl.ds((base + j) * 128, 128)

        def fire_load(j, b):
            pltpu.async_copy(s_hbm.at[in_slot(j)], sb.at[b], lsem.at[b])
            pltpu.async_copy(fc_hbm.at[in_slot(j)], fb.at[b], lsem.at[b])

        def wait_load(j, b):
            pltpu.make_async_copy(s_hbm.at[in_slot(j)], sb.at[b], lsem.at[b]).wait()
            pltpu.make_async_copy(fc_hbm.at[in_slot(j)], fb.at[b], lsem.at[b]).wait()

        def fire_scatter(j, b):
            pltpu.async_copy(sb.at[b], accS.at[idx2.at[b]], ssem.at[b], add=True)
            pltpu.async_copy(fb.at[b], accF.at[idx2.at[b]], ssem.at[b], add=True)

        def wait_scatter(j, b):
            pltpu.make_async_copy(sb.at[b], accS.at[idx2.at[b]], ssem.at[b]).wait()
            pltpu.make_async_copy(fb.at[b], accF.at[idx2.at[b]], ssem.at[b]).wait()

        fire_load(0, 0)

        @pl.loop(0, CPS, step=2)
        def _(j):
            for b in (0, 1):
                jj = j + b

                @pl.when(jj >= 1)
                def _():
                    wait_scatter(jj - 1, 1 - b)

                @pl.when(jj + 1 < CPS)
                def _():
                    fire_load(jj + 1, 1 - b)

                wait_load(jj, b)
                for v in range(0, 128, 16):
                    w = idxs[jj, pl.ds(v, 16)] - lo
                    ok = (w >= 0) & (w < HALF)
                    idx2[b, pl.ds(v, 16)] = jnp.where(ok, w, HALF)
                fire_scatter(jj, b)

        wait_scatter(CPS - 1, 1)
        plsc.subcore_barrier()
        pltpu.sync_copy(accS.at[pl.ds(sid * 313, 313)],
                        sout_hbm.at[pl.ds(lo + sid * 313, 313)])
        pltpu.sync_copy(accF.at[pl.ds(sid * 313, 313)],
                        fcout_hbm.at[pl.ds(lo + sid * 313, 313)])

    return k(s, fc, rowc_s)


def _pre_body(h_ref, Wr_ref, Wc_ref, We2_ref, Wc1_ref, be2_ref, bc1_ref,
              T1_ref, T2_ref, Wec_ref, bec_ref):
    h = h_ref[...]
    T1_ref[...] = jnp.dot(h, Wr_ref[...], precision=_PREC)
    T2_ref[...] = jnp.dot(h, Wc_ref[...], precision=_PREC)
    Wec_ref[...] = jnp.dot(We2_ref[...], Wc1_ref[...], precision=_PREC)
    bec_ref[...] = jnp.dot(be2_ref[...], Wc1_ref[...], precision=_PREC) + bc1_ref[...]


def _node_pre(h, Wr, Wc, We2, Wc1, be2, bc1):
    NB = 1000
    grid = (N // NB,)
    return pl.pallas_call(
        _pre_body,
        grid=grid,
        in_specs=[
            pl.BlockSpec((NB, D), lambda i: (i, 0)),
            pl.BlockSpec((D, D), lambda i: (0, 0)),
            pl.BlockSpec((D, D), lambda i: (0, 0)),
            pl.BlockSpec((D, D), lambda i: (0, 0)),
            pl.BlockSpec((D, D), lambda i: (0, 0)),
            pl.BlockSpec((1, D), lambda i: (0, 0)),
            pl.BlockSpec((1, D), lambda i: (0, 0)),
        ],
        out_specs=[
            pl.BlockSpec((NB, D), lambda i: (i, 0)),
            pl.BlockSpec((NB, D), lambda i: (i, 0)),
            pl.BlockSpec((D, D), lambda i: (0, 0)),
            pl.BlockSpec((1, D), lambda i: (0, 0)),
        ],
        out_shape=[
            jax.ShapeDtypeStruct((N, D), jnp.float32),
            jax.ShapeDtypeStruct((N, D), jnp.float32),
            jax.ShapeDtypeStruct((D, D), jnp.float32),
            jax.ShapeDtypeStruct((1, D), jnp.float32),
        ],
    )(h, Wr, Wc, We2, Wc1, be2, bc1)


def _edge_body(G_ref, RIJ_ref, ea_ref,
               Ws_ref, Wa_ref, be1_ref, Wec_ref, bec_ref, Wc2_ref, bc2_ref,
               s_ref, fc_ref):
    rij = RIJ_ref[...]
    scalar = jnp.sum(rij * rij, axis=1, keepdims=True)
    u = (G_ref[...]
         + scalar * Ws_ref[...]
         + jnp.dot(ea_ref[...], Wa_ref[...], precision=_PREC)
         + be1_ref[...])
    s = _silu(u)
    s_ref[...] = s
    t = jnp.dot(s, Wec_ref[...], precision=_PREC) + bec_ref[...]
    cm = jnp.dot(_silu(t), Wc2_ref[...], precision=_PREC) + bc2_ref[...]
    ones = (lax.broadcasted_iota(jnp.int32, rij.shape, 1) == 3).astype(jnp.float32)
    fc_ref[...] = rij * cm[:, 0:1] + ones


def _edge_mlp(G, RIJ, ea, Ws, Wa, be1, Wec, bec, Wc2, bc2):
    EB = 4096
    EPH = G.shape[0]
    grid = (EPH // EB,)
    return pl.pallas_call(
        _edge_body,
        grid=grid,
        in_specs=[
            pl.BlockSpec((EB, D), lambda i: (i, 0)),
            pl.BlockSpec((EB, 16), lambda i: (i, 0)),
            pl.BlockSpec((EB, 16), lambda i: (i, 0)),
            pl.BlockSpec((1, D), lambda i: (0, 0)),
            pl.BlockSpec((16, D), lambda i: (0, 0)),
            pl.BlockSpec((1, D), lambda i: (0, 0)),
            pl.BlockSpec((D, D), lambda i: (0, 0)),
            pl.BlockSpec((1, D), lambda i: (0, 0)),
            pl.BlockSpec((D, 128), lambda i: (0, 0)),
            pl.BlockSpec((1, 128), lambda i: (0, 0)),
        ],
        out_specs=[
            pl.BlockSpec((EB, D), lambda i: (i, 0)),
            pl.BlockSpec((EB, 16), lambda i: (i, 0)),
        ],
        out_shape=[
            jax.ShapeDtypeStruct((EPH, D), jnp.float32),
            jax.ShapeDtypeStruct((EPH, 16), jnp.float32),
        ],
    )(G, RIJ, ea, Ws, Wa, be1, Wec, bec, Wc2, bc2)


def _post_body(h_ref, S0_ref, S1_ref, FC0_ref, FC1_ref, xp_ref,
               We2_ref, be2_ref, Wn1h_ref, Wn1t_ref, bn1_ref, Wn2_ref, bn2_ref,
               hn_ref, xf_ref):
    h = h_ref[...]
    S = S0_ref[...] + S1_ref[...]
    FC = FC0_ref[...] + FC1_ref[...]
    cnt = FC[:, 3:4]
    tot = jnp.dot(S, We2_ref[...], precision=_PREC) + cnt * be2_ref[...]
    a = (jnp.dot(h, Wn1h_ref[...], precision=_PREC)
         + jnp.dot(tot, Wn1t_ref[...], precision=_PREC) + bn1_ref[...])
    hn_ref[...] = h + jnp.dot(_silu(a), Wn2_ref[...], precision=_PREC) + bn2_ref[...]
    tot_f = jnp.clip(FC / jnp.maximum(cnt, 1.0), -100.0, 100.0)
    xf_ref[...] = xp_ref[...] + tot_f


def _node_post(h, S0, S1, FC0, FC1, xp16, We2, be2, Wn1h, Wn1t, bn1, Wn2, bn2):
    NB = 1000
    grid = (N // NB,)
    return pl.pallas_call(
        _post_body,
        grid=grid,
        in_specs=[
            pl.BlockSpec((NB, D), lambda i: (i, 0)),
            pl.BlockSpec((NB, D), lambda i: (i, 0)),
            pl.BlockSpec((NB, D), lambda i: (i, 0)),
            pl.BlockSpec((NB, 16), lambda i: (i, 0)),
            pl.BlockSpec((NB, 16), lambda i: (i, 0)),
            pl.BlockSpec((NB, 16), lambda i: (i, 0)),
            pl.BlockSpec((D, D), lambda i: (0, 0)),
            pl.BlockSpec((1, D), lambda i: (0, 0)),
            pl.BlockSpec((D, D), lambda i: (0, 0)),
            pl.BlockSpec((D, D), lambda i: (0, 0)),
            pl.BlockSpec((1, D), lambda i: (0, 0)),
            pl.BlockSpec((D, D), lambda i: (0, 0)),
            pl.BlockSpec((1, D), lambda i: (0, 0)),
        ],
        out_specs=[
            pl.BlockSpec((NB, D), lambda i: (i, 0)),
            pl.BlockSpec((NB, 16), lambda i: (i, 0)),
        ],
        out_shape=[
            jax.ShapeDtypeStruct((N, D), jnp.float32),
            jax.ShapeDtypeStruct((N, 16), jnp.float32),
        ],
    )(h, S0, S1, FC0, FC1, xp16, We2, be2, Wn1h, Wn1t, bn1, Wn2, bn2)


@jax.jit
def kernel(x, h, edge_index, edge_attr,
           W_e1, b_e1, W_e2, b_e2,
           W_c1, b_c1, W_c2, b_c2,
           W_n1, b_n1, W_n2, b_n2):
    row = edge_index[0].astype(jnp.int32)
    col = edge_index[1].astype(jnp.int32)
    Ws = W_e1[0:1]
    Wr = W_e1[1:1 + D]
    Wc = W_e1[1 + D:1 + 2 * D]
    Wa = jnp.zeros((16, D), jnp.float32).at[:5].set(W_e1[1 + 2 * D:])
    be1 = b_e1[None, :]
    be2 = b_e2[None, :]
    bc1 = b_c1[None, :]
    bc2 = jnp.zeros((1, 128), jnp.float32).at[0, 0].set(b_c2[0])
    Wc2 = jnp.zeros((D, 128), jnp.float32).at[:, 0:1].set(W_c2)
    Wn1h = W_n1[:D]
    Wn1t = W_n1[D:]
    bn1 = b_n1[None, :]
    bn2 = b_n2[None, :]

    xp16 = jnp.pad(x, ((0, 0), (0, 13)))
    T1, T2, Wec, bec = _node_pre(h, Wr, Wc, W_e2, W_c1, be2, bc1)

    rowc_g = jnp.pad(row.reshape(NCHUNK, 125), ((0, 0), (0, 3)))
    colc_g = jnp.pad(col.reshape(NCHUNK, 125), ((0, 0), (0, 3)))
    rowc_s = jnp.pad(row.reshape(NCHUNK, 125), ((0, 0), (0, 3)),
                     constant_values=N)
    ea16 = jnp.pad(edge_attr.reshape(NCHUNK, 125, 5),
                   ((0, 0), (0, 3), (0, 11))).reshape(EP, 16)

    NH = NCHUNK // 2
    EPH = NH * 128
    souts = []
    fcouts = []
    for hf in (0, 1):
        cs = slice(hf * NH, (hf + 1) * NH)
        es = slice(hf * EPH, (hf + 1) * EPH)
        G, RIJ = _sc_gather(T1, T2, xp16, rowc_g[cs], colc_g[cs], NH)
        s, fc = _edge_mlp(G, RIJ, ea16[es], Ws, Wa, be1,
                          Wec, bec, Wc2, bc2)
        so, fo = _sc_scatter(s, fc, rowc_s[cs], NH)
        souts.append(so)
        fcouts.append(fo)

    hn, xf = _node_post(h, souts[0][:N], souts[1][:N],
                        fcouts[0][:N], fcouts[1][:N], xp16, W_e2, be2,
                        Wn1h, Wn1t, bn1, W_n2, bn2)
    return (xf[:, :3], hn)


# speedup vs baseline: 5.3338x; 1.7791x over previous
import functools
import jax
import jax.numpy as jnp
from jax import lax
from jax.experimental import pallas as pl
from jax.experimental.pallas import tpu as pltpu
from jax.experimental.pallas import tpu_sc as plsc

N = 10000
E = 320000
D = 128

NCHUNK = E // 128
NW = 32
HALF = 5008
ACC_R = HALF + 16
NA = 2 * HALF

_PREC = lax.Precision.HIGHEST
_EPREC = lax.Precision.DEFAULT


def _silu(v):
    return v * jax.nn.sigmoid(v)


def _sc_gather(T1, T2, X16, rowc, colc, nchunk):
    mesh = plsc.VectorSubcoreMesh(core_axis_name="c", subcore_axis_name="s")
    CPW = nchunk // NW
    REM = nchunk - CPW * NW
    CPE = CPW - (CPW % 2)

    @functools.partial(
        pl.kernel, mesh=mesh,
        compiler_params=pltpu.CompilerParams(use_tc_tiling_on_sc=False),
        out_type=[
            jax.ShapeDtypeStruct((nchunk * 128, D), jnp.float32),
            jax.ShapeDtypeStruct((nchunk * 128, 16), jnp.float32),
        ],
        scratch_types=[
            pltpu.VMEM((CPW + 1, 128), jnp.int32),
            pltpu.VMEM((CPW + 1, 128), jnp.int32),
            pltpu.VMEM((2, 128, D), jnp.float32),
            pltpu.VMEM((2, 128, D), jnp.float32),
            pltpu.VMEM((2, 128, 16), jnp.float32),
            pltpu.VMEM((2, 128, 16), jnp.float32),
            pltpu.SemaphoreType.DMA((2,)),
            pltpu.SemaphoreType.DMA((2,)),
        ],
    )
    def k(T1_hbm, T2_hbm, X16_hbm, rowc_hbm, colc_hbm,
          G_hbm, RIJ_hbm,
          idxr, idxc, g1, g2, xr, xc, gsem, wsem):
        wid = lax.axis_index("s") * 2 + lax.axis_index("c")
        base = wid * CPW + jnp.minimum(wid, REM)
        extra = wid < REM
        pltpu.sync_copy(rowc_hbm.at[pl.ds(base, CPW)], idxr.at[pl.ds(0, CPW)])
        pltpu.sync_copy(colc_hbm.at[pl.ds(base, CPW)], idxc.at[pl.ds(0, CPW)])

        @pl.when(extra)
        def _():
            pltpu.sync_copy(rowc_hbm.at[base + CPW], idxr.at[CPW])
            pltpu.sync_copy(colc_hbm.at[base + CPW], idxc.at[CPW])

        def fire_gather(j, b):
            pltpu.async_copy(T1_hbm.at[idxr.at[j]], g1.at[b], gsem.at[b])
            pltpu.async_copy(T2_hbm.at[idxc.at[j]], g2.at[b], gsem.at[b])
            pltpu.async_copy(X16_hbm.at[idxr.at[j]], xr.at[b], gsem.at[b])
            pltpu.async_copy(X16_hbm.at[idxc.at[j]], xc.at[b], gsem.at[b])

        def wait_gather(j, b):
            pltpu.make_async_copy(T1_hbm.at[idxr.at[j]], g1.at[b], gsem.at[b]).wait()
            pltpu.make_async_copy(T2_hbm.at[idxc.at[j]], g2.at[b], gsem.at[b]).wait()
            pltpu.make_async_copy(X16_hbm.at[idxr.at[j]], xr.at[b], gsem.at[b]).wait()
            pltpu.make_async_copy(X16_hbm.at[idxc.at[j]], xc.at[b], gsem.at[b]).wait()

        def out_slot(chunk):
            return pl.ds(chunk * 128, 128)

        def fire_write(j, b):
            chunk = base + j
            pltpu.async_copy(g1.at[b], G_hbm.at[out_slot(chunk)], wsem.at[b])
            pltpu.async_copy(xr.at[b], RIJ_hbm.at[out_slot(chunk)], wsem.at[b])

        def wait_write(j, b):
            chunk = base + j
            pltpu.make_async_copy(g1.at[b], G_hbm.at[out_slot(chunk)], wsem.at[b]).wait()
            pltpu.make_async_copy(xr.at[b], RIJ_hbm.at[out_slot(chunk)], wsem.at[b]).wait()

        def combine(b):
            @pl.loop(0, 128)
            def _(r):
                for v in range(0, D, 16):
                    g1[b, r, pl.ds(v, 16)] = (g1[b, r, pl.ds(v, 16)]
                                              + g2[b, r, pl.ds(v, 16)])
                xr[b, r, pl.ds(0, 16)] = (xr[b, r, pl.ds(0, 16)]
                                          - xc[b, r, pl.ds(0, 16)])

        fire_gather(0, 0)

        @pl.loop(0, CPE, step=2)
        def _(j):
            for b in (0, 1):
                jj = j + b

                @pl.when(jj >= 1)
                def _():
                    wait_write(jj - 1, 1 - b)

                fire_gather(jj + 1, 1 - b)
                wait_gather(jj, b)
                combine(b)
                fire_write(jj, b)

        wait_write(CPE - 1, 1)

        @pl.when(extra)
        def _():
            fire_gather(CPW, 1)

        wait_gather(CPE, 0)
        combine(0)
        fire_write(CPE, 0)

        @pl.when(extra)
        def _():
            wait_write(CPE, 0)
            wait_gather(CPW, 1)
            combine(1)
            fire_write(CPW, 1)
            wait_write(CPW, 1)

        @pl.when(jnp.logical_not(extra))
        def _():
            wait_write(CPE, 0)

    return k(T1, T2, X16, rowc, colc)


def _sc_scatter(s, fc, rowc_s, nchunk):
    mesh = plsc.VectorSubcoreMesh(core_axis_name="c", subcore_axis_name="s")
    CPS = nchunk // 16
    REMS = nchunk - CPS * 16

    @functools.partial(
        pl.kernel, mesh=mesh,
        compiler_params=pltpu.CompilerParams(use_tc_tiling_on_sc=False),
        out_type=[
            jax.ShapeDtypeStruct((NA, D), jnp.float32),
            jax.ShapeDtypeStruct((NA, 16), jnp.float32),
        ],
        scratch_types=[
            pltpu.VMEM((CPS + 1, 128), jnp.int32),
            pltpu.VMEM((2, 128), jnp.int32),
            pltpu.VMEM((2, 128, D), jnp.float32),
            pltpu.VMEM((2, 128, 16), jnp.float32),
            pltpu.VMEM((157, D), jnp.float32),
            pltpu.VMEM((157, 16), jnp.float32),
            pltpu.VMEM_SHARED((ACC_R, D), jnp.float32),
            pltpu.VMEM_SHARED((ACC_R, 16), jnp.float32),
            pltpu.SemaphoreType.DMA((2,)),
            pltpu.SemaphoreType.DMA((2,)),
        ],
    )
    def k(s_hbm, fc_hbm, rowc_hbm, sout_hbm, fcout_hbm,
          idxs, idx2, sb, fb, zs, zf, accS, accF, lsem, ssem):
        cid = lax.axis_index("c")
        sid = lax.axis_index("s")
        lo = cid * HALF
        base = sid * CPS + jnp.minimum(sid, REMS)
        extra = sid < REMS

        @pl.loop(0, 157)
        def _(r):
            @pl.loop(0, D, step=16)
            def _(v):
                zs[r, pl.ds(v, 16)] = jnp.zeros((16,), jnp.float32)
            zf[r, pl.ds(0, 16)] = jnp.zeros((16,), jnp.float32)

        zlo = sid * 314
        pltpu.sync_copy(zs, accS.at[pl.ds(zlo, 157)])
        pltpu.sync_copy(zs, accS.at[pl.ds(zlo + 157, 157)])
        pltpu.sync_copy(zf, accF.at[pl.ds(zlo, 157)])
        pltpu.sync_copy(zf, accF.at[pl.ds(zlo + 157, 157)])
        pltpu.sync_copy(rowc_hbm.at[pl.ds(base, CPS)], idxs.at[pl.ds(0, CPS)])

        @pl.when(extra)
        def _():
            pltpu.sync_copy(rowc_hbm.at[base + CPS], idxs.at[CPS])

        plsc.subcore_barrier()

        def in_slot(j):
            return pl.ds((base + j) * 128, 128)

        def fire_load(j, b):
            pltpu.async_copy(s_hbm.at[in_slot(j)], sb.at[b], lsem.at[b])
            pltpu.async_copy(fc_hbm.at[in_slot(j)], fb.at[b], lsem.at[b])

        def wait_load(j, b):
            pltpu.make_async_copy(s_hbm.at[in_slot(j)], sb.at[b], lsem.at[b]).wait()
            pltpu.make_async_copy(fc_hbm.at[in_slot(j)], fb.at[b], lsem.at[b]).wait()

        def fire_scatter(j, b):
            pltpu.async_copy(sb.at[b], accS.at[idx2.at[b]], ssem.at[b], add=True)
            pltpu.async_copy(fb.at[b], accF.at[idx2.at[b]], ssem.at[b], add=True)

        def wait_scatter(j, b):
            pltpu.make_async_copy(sb.at[b], accS.at[idx2.at[b]], ssem.at[b]).wait()
            pltpu.make_async_copy(fb.at[b], accF.at[idx2.at[b]], ssem.at[b]).wait()

        def mask_idx(jj, b):
            for v in range(0, 128, 16):
                w = idxs[jj, pl.ds(v, 16)] - lo
                ok = (w >= 0) & (w < HALF)
                idx2[b, pl.ds(v, 16)] = jnp.where(ok, w, HALF)

        fire_load(0, 0)

        @pl.loop(0, CPS, step=2)
        def _(j):
            for b in (0, 1):
                jj = j + b

                @pl.when(jj >= 1)
                def _():
                    wait_scatter(jj - 1, 1 - b)

                @pl.when((jj + 1 < CPS) | ((jj + 1 == CPS) & extra))
                def _():
                    fire_load(jj + 1, 1 - b)

                wait_load(jj, b)
                mask_idx(jj, b)
                fire_scatter(jj, b)

        @pl.when(extra)
        def _():
            wait_scatter(CPS - 1, 1)
            wait_load(CPS, 0)
            mask_idx(CPS, 0)
            fire_scatter(CPS, 0)
            wait_scatter(CPS, 0)

        @pl.when(jnp.logical_not(extra))
        def _():
            wait_scatter(CPS - 1, 1)

        plsc.subcore_barrier()
        pltpu.sync_copy(accS.at[pl.ds(sid * 313, 313)],
                        sout_hbm.at[pl.ds(lo + sid * 313, 313)])
        pltpu.sync_copy(accF.at[pl.ds(sid * 313, 313)],
                        fcout_hbm.at[pl.ds(lo + sid * 313, 313)])

    return k(s, fc, rowc_s)


def _pre_body(h_ref, Wr_ref, Wc_ref, We2_ref, Wc1_ref, be2_ref, bc1_ref,
              T1_ref, T2_ref, Wec_ref, bec_ref):
    h = h_ref[...]
    T1_ref[...] = jnp.dot(h, Wr_ref[...], precision=_PREC)
    T2_ref[...] = jnp.dot(h, Wc_ref[...], precision=_PREC)
    Wec_ref[...] = jnp.dot(We2_ref[...], Wc1_ref[...], precision=_PREC)
    bec_ref[...] = jnp.dot(be2_ref[...], Wc1_ref[...], precision=_PREC) + bc1_ref[...]


def _node_pre(h, Wr, Wc, We2, Wc1, be2, bc1):
    NB = 1000
    grid = (N // NB,)
    return pl.pallas_call(
        _pre_body,
        grid=grid,
        in_specs=[
            pl.BlockSpec((NB, D), lambda i: (i, 0)),
            pl.BlockSpec((D, D), lambda i: (0, 0)),
            pl.BlockSpec((D, D), lambda i: (0, 0)),
            pl.BlockSpec((D, D), lambda i: (0, 0)),
            pl.BlockSpec((D, D), lambda i: (0, 0)),
            pl.BlockSpec((1, D), lambda i: (0, 0)),
            pl.BlockSpec((1, D), lambda i: (0, 0)),
        ],
        out_specs=[
            pl.BlockSpec((NB, D), lambda i: (i, 0)),
            pl.BlockSpec((NB, D), lambda i: (i, 0)),
            pl.BlockSpec((D, D), lambda i: (0, 0)),
            pl.BlockSpec((1, D), lambda i: (0, 0)),
        ],
        out_shape=[
            jax.ShapeDtypeStruct((N, D), jnp.float32),
            jax.ShapeDtypeStruct((N, D), jnp.float32),
            jax.ShapeDtypeStruct((D, D), jnp.float32),
            jax.ShapeDtypeStruct((1, D), jnp.float32),
        ],
    )(h, Wr, Wc, We2, Wc1, be2, bc1)


def _edge_body(G_ref, RIJ_ref, ea_ref,
               Ws_ref, Wa_ref, be1_ref, Wec_ref, bec_ref, Wc2_ref, bc2_ref,
               s_ref, fc_ref):
    rij = RIJ_ref[...]
    scalar = jnp.sum(rij * rij, axis=1, keepdims=True)
    u = (G_ref[...]
         + scalar * Ws_ref[...]
         + jnp.dot(ea_ref[...], Wa_ref[...], precision=_EPREC)
         + be1_ref[...])
    s = _silu(u)
    s_ref[...] = s
    t = jnp.dot(s, Wec_ref[...], precision=_EPREC) + bec_ref[...]
    cm = jnp.dot(_silu(t), Wc2_ref[...], precision=_EPREC) + bc2_ref[...]
    ones = (lax.broadcasted_iota(jnp.int32, rij.shape, 1) == 3).astype(jnp.float32)
    fc_ref[...] = rij * cm[:, 0:1] + ones


def _edge_mlp(G, RIJ, ea_full, hf, Ws, Wa, be1, Wec, bec, Wc2, bc2):
    EB = 4000
    EPH = G.shape[0]
    nblk = EPH // EB
    off = hf * nblk
    grid = (nblk,)
    return pl.pallas_call(
        _edge_body,
        grid=grid,
        in_specs=[
            pl.BlockSpec((EB, D), lambda i: (i, 0)),
            pl.BlockSpec((EB, 16), lambda i: (i, 0)),
            pl.BlockSpec((EB, 5), lambda i: (i + off, 0)),
            pl.BlockSpec((1, D), lambda i: (0, 0)),
            pl.BlockSpec((5, D), lambda i: (0, 0)),
            pl.BlockSpec((1, D), lambda i: (0, 0)),
            pl.BlockSpec((D, D), lambda i: (0, 0)),
            pl.BlockSpec((1, D), lambda i: (0, 0)),
            pl.BlockSpec((D, 128), lambda i: (0, 0)),
            pl.BlockSpec((1, 128), lambda i: (0, 0)),
        ],
        out_specs=[
            pl.BlockSpec((EB, D), lambda i: (i, 0)),
            pl.BlockSpec((EB, 16), lambda i: (i, 0)),
        ],
        out_shape=[
            jax.ShapeDtypeStruct((EPH, D), jnp.float32),
            jax.ShapeDtypeStruct((EPH, 16), jnp.float32),
        ],
    )(G, RIJ, ea_full, Ws, Wa, be1, Wec, bec, Wc2, bc2)


def _post_body(h_ref, S0_ref, S1_ref, FC0_ref, FC1_ref, xp_ref,
               We2_ref, be2_ref, Wn1h_ref, Wn1t_ref, bn1_ref, Wn2_ref, bn2_ref,
               hn_ref, xf_ref):
    h = h_ref[...]
    S = S0_ref[...] + S1_ref[...]
    FC = FC0_ref[...] + FC1_ref[...]
    cnt = FC[:, 3:4]
    tot = jnp.dot(S, We2_ref[...], precision=_PREC) + cnt * be2_ref[...]
    a = (jnp.dot(h, Wn1h_ref[...], precision=_PREC)
         + jnp.dot(tot, Wn1t_ref[...], precision=_PREC) + bn1_ref[...])
    hn_ref[...] = h + jnp.dot(_silu(a), Wn2_ref[...], precision=_PREC) + bn2_ref[...]
    tot_f = jnp.clip(FC / jnp.maximum(cnt, 1.0), -100.0, 100.0)
    xf_ref[...] = xp_ref[...] + tot_f


def _node_post(h, S0, S1, FC0, FC1, xp16, We2, be2, Wn1h, Wn1t, bn1, Wn2, bn2):
    NB = 1000
    grid = (N // NB,)
    return pl.pallas_call(
        _post_body,
        grid=grid,
        in_specs=[
            pl.BlockSpec((NB, D), lambda i: (i, 0)),
            pl.BlockSpec((NB, D), lambda i: (i, 0)),
            pl.BlockSpec((NB, D), lambda i: (i, 0)),
            pl.BlockSpec((NB, 16), lambda i: (i, 0)),
            pl.BlockSpec((NB, 16), lambda i: (i, 0)),
            pl.BlockSpec((NB, 16), lambda i: (i, 0)),
            pl.BlockSpec((D, D), lambda i: (0, 0)),
            pl.BlockSpec((1, D), lambda i: (0, 0)),
            pl.BlockSpec((D, D), lambda i: (0, 0)),
            pl.BlockSpec((D, D), lambda i: (0, 0)),
            pl.BlockSpec((1, D), lambda i: (0, 0)),
            pl.BlockSpec((D, D), lambda i: (0, 0)),
            pl.BlockSpec((1, D), lambda i: (0, 0)),
        ],
        out_specs=[
            pl.BlockSpec((NB, D), lambda i: (i, 0)),
            pl.BlockSpec((NB, 16), lambda i: (i, 0)),
        ],
        out_shape=[
            jax.ShapeDtypeStruct((N, D), jnp.float32),
            jax.ShapeDtypeStruct((N, 16), jnp.float32),
        ],
    )(h, S0, S1, FC0, FC1, xp16, We2, be2, Wn1h, Wn1t, bn1, Wn2, bn2)


@jax.jit
def kernel(x, h, edge_index, edge_attr,
           W_e1, b_e1, W_e2, b_e2,
           W_c1, b_c1, W_c2, b_c2,
           W_n1, b_n1, W_n2, b_n2):
    row = edge_index[0].astype(jnp.int32)
    col = edge_index[1].astype(jnp.int32)
    Ws = W_e1[0:1]
    Wr = W_e1[1:1 + D]
    Wc = W_e1[1 + D:1 + 2 * D]
    Wa = W_e1[1 + 2 * D:]
    be1 = b_e1[None, :]
    be2 = b_e2[None, :]
    bc1 = b_c1[None, :]
    bc2 = jnp.zeros((1, 128), jnp.float32).at[0, 0].set(b_c2[0])
    Wc2 = jnp.zeros((D, 128), jnp.float32).at[:, 0:1].set(W_c2)
    Wn1h = W_n1[:D]
    Wn1t = W_n1[D:]
    bn1 = b_n1[None, :]
    bn2 = b_n2[None, :]

    xp16 = jnp.pad(x, ((0, 0), (0, 13)))
    T1, T2, Wec, bec = _node_pre(h, Wr, Wc, W_e2, W_c1, be2, bc1)

    rowc = row.reshape(NCHUNK, 128)
    colc = col.reshape(NCHUNK, 128)

    NH = NCHUNK // 2
    souts = []
    fcouts = []
    for hf in (0, 1):
        cs = slice(hf * NH, (hf + 1) * NH)
        G, RIJ = _sc_gather(T1, T2, xp16, rowc[cs], colc[cs], NH)
        s, fc = _edge_mlp(G, RIJ, edge_attr, hf, Ws, Wa, be1,
                          Wec, bec, Wc2, bc2)
        so, fo = _sc_scatter(s, fc, rowc[cs], NH)
        souts.append(so)
        fcouts.append(fo)

    hn, xf = _node_post(h, souts[0][:N], souts[1][:N],
                        fcouts[0][:N], fcouts[1][:N], xp16, W_e2, be2,
                        Wn1h, Wn1t, bn1, W_n2, bn2)
    return (xf[:, :3], hn)


# speedup vs baseline: 5.5720x; 1.0447x over previous
import functools
import jax
import jax.numpy as jnp
from jax import lax
from jax.experimental import pallas as pl
from jax.experimental.pallas import tpu as pltpu
from jax.experimental.pallas import tpu_sc as plsc

N = 10000
E = 320000
D = 128

NCHUNK = E // 128
NW = 32
HALF = 5008
ACC_R = HALF + 16
NA = 2 * HALF

_PREC = lax.Precision.HIGHEST
_EPREC = lax.Precision.DEFAULT


def _silu(v):
    return v * jax.nn.sigmoid(v)


def _sc_gather(T1, T2, X16, rowc, colc, nchunk):
    mesh = plsc.VectorSubcoreMesh(core_axis_name="c", subcore_axis_name="s")
    CPW = nchunk // NW
    REM = nchunk - CPW * NW
    assert CPW % 2 == 0

    @functools.partial(
        pl.kernel, mesh=mesh,
        compiler_params=pltpu.CompilerParams(use_tc_tiling_on_sc=False),
        out_type=[
            jax.ShapeDtypeStruct((nchunk * 128, D), jnp.float32),
            jax.ShapeDtypeStruct((nchunk * 128, 16), jnp.float32),
        ],
        scratch_types=[
            pltpu.VMEM((CPW + 1, 128), jnp.int32),
            pltpu.VMEM((CPW + 1, 128), jnp.int32),
            pltpu.VMEM((2, 128, D), jnp.float32),
            pltpu.VMEM((2, 128, D), jnp.float32),
            pltpu.VMEM((2, 128, 16), jnp.float32),
            pltpu.VMEM((2, 128, 16), jnp.float32),
            pltpu.SemaphoreType.DMA((2,)),
            pltpu.SemaphoreType.DMA((2,)),
        ],
    )
    def k(T1_hbm, T2_hbm, X16_hbm, rowc_hbm, colc_hbm,
          G_hbm, RIJ_hbm,
          idxr, idxc, g1, g2, xr, xc, gsem, wsem):
        wid = lax.axis_index("s") * 2 + lax.axis_index("c")
        base = wid * CPW + jnp.minimum(wid, REM)
        extra = wid < REM
        pltpu.sync_copy(rowc_hbm.at[pl.ds(base, CPW)], idxr.at[pl.ds(0, CPW)])
        pltpu.sync_copy(colc_hbm.at[pl.ds(base, CPW)], idxc.at[pl.ds(0, CPW)])

        @pl.when(extra)
        def _():
            pltpu.sync_copy(rowc_hbm.at[base + CPW], idxr.at[CPW])
            pltpu.sync_copy(colc_hbm.at[base + CPW], idxc.at[CPW])

        def fire_gather(j, b):
            pltpu.async_copy(T1_hbm.at[idxr.at[j]], g1.at[b], gsem.at[b])
            pltpu.async_copy(T2_hbm.at[idxc.at[j]], g2.at[b], gsem.at[b])
            pltpu.async_copy(X16_hbm.at[idxr.at[j]], xr.at[b], gsem.at[b])
            pltpu.async_copy(X16_hbm.at[idxc.at[j]], xc.at[b], gsem.at[b])

        def wait_gather(j, b):
            pltpu.make_async_copy(T1_hbm.at[idxr.at[j]], g1.at[b], gsem.at[b]).wait()
            pltpu.make_async_copy(T2_hbm.at[idxc.at[j]], g2.at[b], gsem.at[b]).wait()
            pltpu.make_async_copy(X16_hbm.at[idxr.at[j]], xr.at[b], gsem.at[b]).wait()
            pltpu.make_async_copy(X16_hbm.at[idxc.at[j]], xc.at[b], gsem.at[b]).wait()

        def out_slot(chunk):
            return pl.ds(chunk * 128, 128)

        def fire_write(j, b):
            chunk = base + j
            pltpu.async_copy(g1.at[b], G_hbm.at[out_slot(chunk)], wsem.at[b])
            pltpu.async_copy(xr.at[b], RIJ_hbm.at[out_slot(chunk)], wsem.at[b])

        def wait_write(j, b):
            chunk = base + j
            pltpu.make_async_copy(g1.at[b], G_hbm.at[out_slot(chunk)], wsem.at[b]).wait()
            pltpu.make_async_copy(xr.at[b], RIJ_hbm.at[out_slot(chunk)], wsem.at[b]).wait()

        def combine(b):
            @pl.loop(0, 128)
            def _(r):
                for v in range(0, D, 16):
                    g1[b, r, pl.ds(v, 16)] = (g1[b, r, pl.ds(v, 16)]
                                              + g2[b, r, pl.ds(v, 16)])
                xr[b, r, pl.ds(0, 16)] = (xr[b, r, pl.ds(0, 16)]
                                          - xc[b, r, pl.ds(0, 16)])

        fire_gather(0, 0)

        @pl.loop(0, CPW, step=2)
        def _(j):
            for b in (0, 1):
                jj = j + b

                @pl.when(jj >= 1)
                def _():
                    wait_write(jj - 1, 1 - b)

                @pl.when((jj + 1 < CPW) | extra)
                def _():
                    fire_gather(jj + 1, 1 - b)

                wait_gather(jj, b)
                combine(b)
                fire_write(jj, b)

        wait_write(CPW - 1, 1)

        @pl.when(extra)
        def _():
            wait_gather(CPW, 0)
            combine(0)
            fire_write(CPW, 0)
            wait_write(CPW, 0)

    return k(T1, T2, X16, rowc, colc)


def _sc_scatter(s, fc, rowc_s, nchunk):
    mesh = plsc.VectorSubcoreMesh(core_axis_name="c", subcore_axis_name="s")
    CPS = nchunk // 16
    REMS = nchunk - CPS * 16

    @functools.partial(
        pl.kernel, mesh=mesh,
        compiler_params=pltpu.CompilerParams(use_tc_tiling_on_sc=False),
        out_type=[
            jax.ShapeDtypeStruct((NA, D), jnp.float32),
            jax.ShapeDtypeStruct((NA, 16), jnp.float32),
        ],
        scratch_types=[
            pltpu.VMEM((CPS + 1, 128), jnp.int32),
            pltpu.VMEM((2, 128), jnp.int32),
            pltpu.VMEM((2, 128, D), jnp.float32),
            pltpu.VMEM((2, 128, 16), jnp.float32),
            pltpu.VMEM((157, D), jnp.float32),
            pltpu.VMEM((157, 16), jnp.float32),
            pltpu.VMEM_SHARED((ACC_R, D), jnp.float32),
            pltpu.VMEM_SHARED((ACC_R, 16), jnp.float32),
            pltpu.SemaphoreType.DMA((2,)),
            pltpu.SemaphoreType.DMA((2,)),
        ],
    )
    def k(s_hbm, fc_hbm, rowc_hbm, sout_hbm, fcout_hbm,
          idxs, idx2, sb, fb, zs, zf, accS, accF, lsem, ssem):
        cid = lax.axis_index("c")
        sid = lax.axis_index("s")
        lo = cid * HALF
        base = sid * CPS + jnp.minimum(sid, REMS)
        extra = sid < REMS

        @pl.loop(0, 157)
        def _(r):
            @pl.loop(0, D, step=16)
            def _(v):
                zs[r, pl.ds(v, 16)] = jnp.zeros((16,), jnp.float32)
            zf[r, pl.ds(0, 16)] = jnp.zeros((16,), jnp.float32)

        zlo = sid * 314
        pltpu.sync_copy(zs, accS.at[pl.ds(zlo, 157)])
        pltpu.sync_copy(zs, accS.at[pl.ds(zlo + 157, 157)])
        pltpu.sync_copy(zf, accF.at[pl.ds(zlo, 157)])
        pltpu.sync_copy(zf, accF.at[pl.ds(zlo + 157, 157)])
        pltpu.sync_copy(rowc_hbm.at[pl.ds(base, CPS)], idxs.at[pl.ds(0, CPS)])

        @pl.when(extra)
        def _():
            pltpu.sync_copy(rowc_hbm.at[base + CPS], idxs.at[CPS])

        plsc.subcore_barrier()

        def in_slot(j):
            return pl.ds((base + j) * 128, 128)

        def fire_load(j, b):
            pltpu.async_copy(s_hbm.at[in_slot(j)], sb.at[b], lsem.at[b])
            pltpu.async_copy(fc_hbm.at[in_slot(j)], fb.at[b], lsem.at[b])

        def wait_load(j, b):
            pltpu.make_async_copy(s_hbm.at[in_slot(j)], sb.at[b], lsem.at[b]).wait()
            pltpu.make_async_copy(fc_hbm.at[in_slot(j)], fb.at[b], lsem.at[b]).wait()

        def fire_scatter(j, b):
            pltpu.async_copy(sb.at[b], accS.at[idx2.at[b]], ssem.at[b], add=True)
            pltpu.async_copy(fb.at[b], accF.at[idx2.at[b]], ssem.at[b], add=True)

        def wait_scatter(j, b):
            pltpu.make_async_copy(sb.at[b], accS.at[idx2.at[b]], ssem.at[b]).wait()
            pltpu.make_async_copy(fb.at[b], accF.at[idx2.at[b]], ssem.at[b]).wait()

        def mask_idx(jj, b):
            for v in range(0, 128, 16):
                w = idxs[jj, pl.ds(v, 16)] - lo
                ok = (w >= 0) & (w < HALF)
                idx2[b, pl.ds(v, 16)] = jnp.where(ok, w, HALF)

        fire_load(0, 0)

        @pl.loop(0, CPS, step=2)
        def _(j):
            for b in (0, 1):
                jj = j + b

                @pl.when(jj >= 1)
                def _():
                    wait_scatter(jj - 1, 1 - b)

                @pl.when((jj + 1 < CPS) | ((jj + 1 == CPS) & extra))
                def _():
                    fire_load(jj + 1, 1 - b)

                wait_load(jj, b)
                mask_idx(jj, b)
                fire_scatter(jj, b)

        @pl.when(extra)
        def _():
            wait_scatter(CPS - 1, 1)
            wait_load(CPS, 0)
            mask_idx(CPS, 0)
            fire_scatter(CPS, 0)
            wait_scatter(CPS, 0)

        @pl.when(jnp.logical_not(extra))
        def _():
            wait_scatter(CPS - 1, 1)

        plsc.subcore_barrier()
        pltpu.sync_copy(accS.at[pl.ds(sid * 313, 313)],
                        sout_hbm.at[pl.ds(lo + sid * 313, 313)])
        pltpu.sync_copy(accF.at[pl.ds(sid * 313, 313)],
                        fcout_hbm.at[pl.ds(lo + sid * 313, 313)])

    return k(s, fc, rowc_s)


def _pre_body(h_ref, Wr_ref, Wc_ref, We2_ref, Wc1_ref, be2_ref, bc1_ref,
              T1_ref, T2_ref, Wec_ref, bec_ref):
    h = h_ref[...]
    T1_ref[...] = jnp.dot(h, Wr_ref[...], precision=_PREC)
    T2_ref[...] = jnp.dot(h, Wc_ref[...], precision=_PREC)
    Wec_ref[...] = jnp.dot(We2_ref[...], Wc1_ref[...], precision=_PREC)
    bec_ref[...] = jnp.dot(be2_ref[...], Wc1_ref[...], precision=_PREC) + bc1_ref[...]


def _node_pre(h, Wr, Wc, We2, Wc1, be2, bc1):
    NB = 1000
    grid = (N // NB,)
    return pl.pallas_call(
        _pre_body,
        grid=grid,
        in_specs=[
            pl.BlockSpec((NB, D), lambda i: (i, 0)),
            pl.BlockSpec((D, D), lambda i: (0, 0)),
            pl.BlockSpec((D, D), lambda i: (0, 0)),
            pl.BlockSpec((D, D), lambda i: (0, 0)),
            pl.BlockSpec((D, D), lambda i: (0, 0)),
            pl.BlockSpec((1, D), lambda i: (0, 0)),
            pl.BlockSpec((1, D), lambda i: (0, 0)),
        ],
        out_specs=[
            pl.BlockSpec((NB, D), lambda i: (i, 0)),
            pl.BlockSpec((NB, D), lambda i: (i, 0)),
            pl.BlockSpec((D, D), lambda i: (0, 0)),
            pl.BlockSpec((1, D), lambda i: (0, 0)),
        ],
        out_shape=[
            jax.ShapeDtypeStruct((N, D), jnp.float32),
            jax.ShapeDtypeStruct((N, D), jnp.float32),
            jax.ShapeDtypeStruct((D, D), jnp.float32),
            jax.ShapeDtypeStruct((1, D), jnp.float32),
        ],
    )(h, Wr, Wc, We2, Wc1, be2, bc1)


def _edge_body(G_ref, RIJ_ref, ea_ref,
               Ws_ref, Wa_ref, be1_ref, Wec_ref, bec_ref, Wc2_ref, bc2_ref,
               s_ref, fc_ref):
    rij = RIJ_ref[...]
    scalar = jnp.sum(rij * rij, axis=1, keepdims=True)
    u = (G_ref[...]
         + scalar * Ws_ref[...]
         + jnp.dot(ea_ref[...], Wa_ref[...], precision=_EPREC)
         + be1_ref[...])
    s = _silu(u)
    s_ref[...] = s
    t = jnp.dot(s, Wec_ref[...], precision=_EPREC) + bec_ref[...]
    cm = jnp.dot(_silu(t), Wc2_ref[...], precision=_EPREC) + bc2_ref[...]
    ones = (lax.broadcasted_iota(jnp.int32, rij.shape, 1) == 3).astype(jnp.float32)
    fc_ref[...] = rij * cm[:, 0:1] + ones


def _edge_mlp(G, RIJ, ea, Ws, Wa, be1, Wec, bec, Wc2, bc2):
    EPH = G.shape[0]
    nblk = 20
    EB = EPH // nblk
    grid = (nblk,)
    return pl.pallas_call(
        _edge_body,
        grid=grid,
        in_specs=[
            pl.BlockSpec((EB, D), lambda i: (i, 0)),
            pl.BlockSpec((EB, 16), lambda i: (i, 0)),
            pl.BlockSpec((EB, 5), lambda i: (i, 0)),
            pl.BlockSpec((1, D), lambda i: (0, 0)),
            pl.BlockSpec((5, D), lambda i: (0, 0)),
            pl.BlockSpec((1, D), lambda i: (0, 0)),
            pl.BlockSpec((D, D), lambda i: (0, 0)),
            pl.BlockSpec((1, D), lambda i: (0, 0)),
            pl.BlockSpec((D, 128), lambda i: (0, 0)),
            pl.BlockSpec((1, 128), lambda i: (0, 0)),
        ],
        out_specs=[
            pl.BlockSpec((EB, D), lambda i: (i, 0)),
            pl.BlockSpec((EB, 16), lambda i: (i, 0)),
        ],
        out_shape=[
            jax.ShapeDtypeStruct((EPH, D), jnp.float32),
            jax.ShapeDtypeStruct((EPH, 16), jnp.float32),
        ],
    )(G, RIJ, ea, Ws, Wa, be1, Wec, bec, Wc2, bc2)


def _post_body(h_ref, S0_ref, S1_ref, S2_ref, S3_ref,
               FC0_ref, FC1_ref, FC2_ref, FC3_ref, xp_ref,
               We2_ref, be2_ref, Wn1h_ref, Wn1t_ref, bn1_ref, Wn2_ref, bn2_ref,
               hn_ref, xf_ref):
    h = h_ref[...]
    S = (S0_ref[...] + S1_ref[...]) + (S2_ref[...] + S3_ref[...])
    FC = (FC0_ref[...] + FC1_ref[...]) + (FC2_ref[...] + FC3_ref[...])
    cnt = FC[:, 3:4]
    tot = jnp.dot(S, We2_ref[...], precision=_PREC) + cnt * be2_ref[...]
    a = (jnp.dot(h, Wn1h_ref[...], precision=_PREC)
         + jnp.dot(tot, Wn1t_ref[...], precision=_PREC) + bn1_ref[...])
    hn_ref[...] = h + jnp.dot(_silu(a), Wn2_ref[...], precision=_PREC) + bn2_ref[...]
    tot_f = jnp.clip(FC / jnp.maximum(cnt, 1.0), -100.0, 100.0)
    xf_ref[...] = xp_ref[...] + tot_f


def _node_post(h, Ss, FCs, xp16, We2, be2, Wn1h, Wn1t, bn1, Wn2, bn2):
    NB = 1000
    grid = (N // NB,)
    return pl.pallas_call(
        _post_body,
        grid=grid,
        in_specs=[
            pl.BlockSpec((NB, D), lambda i: (i, 0)),
            pl.BlockSpec((NB, D), lambda i: (i, 0)),
            pl.BlockSpec((NB, D), lambda i: (i, 0)),
            pl.BlockSpec((NB, D), lambda i: (i, 0)),
            pl.BlockSpec((NB, D), lambda i: (i, 0)),
            pl.BlockSpec((NB, 16), lambda i: (i, 0)),
            pl.BlockSpec((NB, 16), lambda i: (i, 0)),
            pl.BlockSpec((NB, 16), lambda i: (i, 0)),
            pl.BlockSpec((NB, 16), lambda i: (i, 0)),
            pl.BlockSpec((NB, 16), lambda i: (i, 0)),
            pl.BlockSpec((D, D), lambda i: (0, 0)),
            pl.BlockSpec((1, D), lambda i: (0, 0)),
            pl.BlockSpec((D, D), lambda i: (0, 0)),
            pl.BlockSpec((D, D), lambda i: (0, 0)),
            pl.BlockSpec((1, D), lambda i: (0, 0)),
            pl.BlockSpec((D, D), lambda i: (0, 0)),
            pl.BlockSpec((1, D), lambda i: (0, 0)),
        ],
        out_specs=[
            pl.BlockSpec((NB, D), lambda i: (i, 0)),
            pl.BlockSpec((NB, 16), lambda i: (i, 0)),
        ],
        out_shape=[
            jax.ShapeDtypeStruct((N, D), jnp.float32),
            jax.ShapeDtypeStruct((N, 16), jnp.float32),
        ],
    )(h, *Ss, *FCs, xp16, We2, be2, Wn1h, Wn1t, bn1, Wn2, bn2)


@jax.jit
def kernel(x, h, edge_index, edge_attr,
           W_e1, b_e1, W_e2, b_e2,
           W_c1, b_c1, W_c2, b_c2,
           W_n1, b_n1, W_n2, b_n2):
    row = edge_index[0].astype(jnp.int32)
    col = edge_index[1].astype(jnp.int32)
    Ws = W_e1[0:1]
    Wr = W_e1[1:1 + D]
    Wc = W_e1[1 + D:1 + 2 * D]
    Wa = W_e1[1 + 2 * D:]
    be1 = b_e1[None, :]
    be2 = b_e2[None, :]
    bc1 = b_c1[None, :]
    bc2 = jnp.zeros((1, 128), jnp.float32).at[0, 0].set(b_c2[0])
    Wc2 = jnp.zeros((D, 128), jnp.float32).at[:, 0:1].set(W_c2)
    Wn1h = W_n1[:D]
    Wn1t = W_n1[D:]
    bn1 = b_n1[None, :]
    bn2 = b_n2[None, :]

    xp16 = jnp.pad(x, ((0, 0), (0, 13)))
    T1, T2, Wec, bec = _node_pre(h, Wr, Wc, W_e2, W_c1, be2, bc1)

    rowc = row.reshape(NCHUNK, 128)
    colc = col.reshape(NCHUNK, 128)

    phases = [(0, 640), (640, 640), (1280, 640), (1920, 580)]
    souts = []
    fcouts = []
    for c0, nc in phases:
        cs = slice(c0, c0 + nc)
        es = slice(c0 * 128, (c0 + nc) * 128)
        G, RIJ = _sc_gather(T1, T2, xp16, rowc[cs], colc[cs], nc)
        s, fc = _edge_mlp(G, RIJ, edge_attr[es], Ws, Wa, be1,
                          Wec, bec, Wc2, bc2)
        so, fo = _sc_scatter(s, fc, rowc[cs], nc)
        souts.append(so[:N])
        fcouts.append(fo[:N])

    hn, xf = _node_post(h, souts, fcouts, xp16, W_e2, be2,
                        Wn1h, Wn1t, bn1, W_n2, bn2)
    return (xf[:, :3], hn)


# speedup vs baseline: 5.9793x; 1.0731x over previous
import functools
import jax
import jax.numpy as jnp
from jax import lax
from jax.experimental import pallas as pl
from jax.experimental.pallas import tpu as pltpu
from jax.experimental.pallas import tpu_sc as plsc

N = 10000
E = 320000
D = 128

NCHUNK = E // 128
NW = 32
HALF = 5008
ACC_R = HALF + 16
NA = 2 * HALF

_PREC = lax.Precision.HIGHEST
_EPREC = lax.Precision.DEFAULT


def _silu(v):
    return v * jax.nn.sigmoid(v)


def _sc_gather(T1, T2, X16, rowc, colc, nchunk):
    mesh = plsc.VectorSubcoreMesh(core_axis_name="c", subcore_axis_name="s")
    CPW = nchunk // NW
    REM = nchunk - CPW * NW
    assert CPW % 2 == 0

    @functools.partial(
        pl.kernel, mesh=mesh,
        compiler_params=pltpu.CompilerParams(use_tc_tiling_on_sc=False),
        out_type=[
            jax.ShapeDtypeStruct((nchunk * 128, D), jnp.float32),
            jax.ShapeDtypeStruct((nchunk * 128, 16), jnp.float32),
        ],
        scratch_types=[
            pltpu.VMEM((CPW + 1, 128), jnp.int32),
            pltpu.VMEM((CPW + 1, 128), jnp.int32),
            pltpu.VMEM((2, 128, D), jnp.float32),
            pltpu.VMEM((2, 128, D), jnp.float32),
            pltpu.VMEM((2, 128, 16), jnp.float32),
            pltpu.VMEM((2, 128, 16), jnp.float32),
            pltpu.SemaphoreType.DMA((2,)),
            pltpu.SemaphoreType.DMA((2,)),
        ],
    )
    def k(T1_hbm, T2_hbm, X16_hbm, rowc_hbm, colc_hbm,
          G_hbm, RIJ_hbm,
          idxr, idxc, g1, g2, xr, xc, gsem, wsem):
        wid = lax.axis_index("s") * 2 + lax.axis_index("c")
        base = wid * CPW + jnp.minimum(wid, REM)
        extra = wid < REM
        pltpu.sync_copy(rowc_hbm.at[pl.ds(base, CPW)], idxr.at[pl.ds(0, CPW)])
        pltpu.sync_copy(colc_hbm.at[pl.ds(base, CPW)], idxc.at[pl.ds(0, CPW)])

        @pl.when(extra)
        def _():
            pltpu.sync_copy(rowc_hbm.at[base + CPW], idxr.at[CPW])
            pltpu.sync_copy(colc_hbm.at[base + CPW], idxc.at[CPW])

        def fire_gather(j, b):
            pltpu.async_copy(T1_hbm.at[idxr.at[j]], g1.at[b], gsem.at[b])
            pltpu.async_copy(T2_hbm.at[idxc.at[j]], g2.at[b], gsem.at[b])
            pltpu.async_copy(X16_hbm.at[idxr.at[j]], xr.at[b], gsem.at[b])
            pltpu.async_copy(X16_hbm.at[idxc.at[j]], xc.at[b], gsem.at[b])

        def wait_gather(j, b):
            pltpu.make_async_copy(T1_hbm.at[idxr.at[j]], g1.at[b], gsem.at[b]).wait()
            pltpu.make_async_copy(T2_hbm.at[idxc.at[j]], g2.at[b], gsem.at[b]).wait()
            pltpu.make_async_copy(X16_hbm.at[idxr.at[j]], xr.at[b], gsem.at[b]).wait()
            pltpu.make_async_copy(X16_hbm.at[idxc.at[j]], xc.at[b], gsem.at[b]).wait()

        def out_slot(chunk):
            return pl.ds(chunk * 128, 128)

        def fire_write(j, b):
            chunk = base + j
            pltpu.async_copy(g1.at[b], G_hbm.at[out_slot(chunk)], wsem.at[b])
            pltpu.async_copy(xr.at[b], RIJ_hbm.at[out_slot(chunk)], wsem.at[b])

        def wait_write(j, b):
            chunk = base + j
            pltpu.make_async_copy(g1.at[b], G_hbm.at[out_slot(chunk)], wsem.at[b]).wait()
            pltpu.make_async_copy(xr.at[b], RIJ_hbm.at[out_slot(chunk)], wsem.at[b]).wait()

        def combine(b):
            @pl.loop(0, 128)
            def _(r):
                for v in range(0, D, 16):
                    g1[b, r, pl.ds(v, 16)] = (g1[b, r, pl.ds(v, 16)]
                                              + g2[b, r, pl.ds(v, 16)])
                xr[b, r, pl.ds(0, 16)] = (xr[b, r, pl.ds(0, 16)]
                                          - xc[b, r, pl.ds(0, 16)])

        fire_gather(0, 0)

        @pl.loop(0, CPW, step=2)
        def _(j):
            for b in (0, 1):
                jj = j + b

                @pl.when(jj >= 1)
                def _():
                    wait_write(jj - 1, 1 - b)

                @pl.when((jj + 1 < CPW) | extra)
                def _():
                    fire_gather(jj + 1, 1 - b)

                wait_gather(jj, b)
                combine(b)
                fire_write(jj, b)

        wait_write(CPW - 1, 1)

        @pl.when(extra)
        def _():
            wait_gather(CPW, 0)
            combine(0)
            fire_write(CPW, 0)
            wait_write(CPW, 0)

    return k(T1, T2, X16, rowc, colc)


def _sc_scatter(s, fc, rowc_s, nchunk):
    mesh = plsc.VectorSubcoreMesh(core_axis_name="c", subcore_axis_name="s")
    CPS = nchunk // 16
    REMS = nchunk - CPS * 16
    HD = D // 2

    @functools.partial(
        pl.kernel, mesh=mesh,
        compiler_params=pltpu.CompilerParams(use_tc_tiling_on_sc=False),
        out_type=[
            jax.ShapeDtypeStruct((NA, D), jnp.float32),
            jax.ShapeDtypeStruct((NA, 16), jnp.float32),
        ],
        scratch_types=[
            pltpu.VMEM((CPS + 1, 128), jnp.int32),
            pltpu.VMEM((2, 128, HD), jnp.float32),
            pltpu.VMEM((2, 128, 16), jnp.float32),
            pltpu.VMEM((313, HD), jnp.float32),
            pltpu.VMEM((313, 16), jnp.float32),
            pltpu.VMEM_SHARED((NA, HD), jnp.float32),
            pltpu.VMEM_SHARED((NA, 16), jnp.float32),
            pltpu.SemaphoreType.DMA((2,)),
            pltpu.SemaphoreType.DMA((2,)),
            pltpu.SemaphoreType.DMA((2,)),
            pltpu.SemaphoreType.DMA((2,)),
        ],
    )
    def k(s_hbm, fc_hbm, rowc_hbm, sout_hbm, fcout_hbm,
          idxs, sb, fb, zs, zf, accS, accF, lsem, fsem, ssem, tsem):
        cid = lax.axis_index("c")
        sid = lax.axis_index("s")
        on0 = cid == 0
        base = sid * CPS + jnp.minimum(sid, REMS)
        extra = sid < REMS

        @pl.loop(0, 313)
        def _(r):
            @pl.loop(0, HD, step=16)
            def _(v):
                zs[r, pl.ds(v, 16)] = jnp.zeros((16,), jnp.float32)
            zf[r, pl.ds(0, 16)] = jnp.zeros((16,), jnp.float32)

        zlo = sid * 626
        pltpu.sync_copy(zs, accS.at[pl.ds(zlo, 313)])
        pltpu.sync_copy(zs, accS.at[pl.ds(zlo + 313, 313)])

        @pl.when(on0)
        def _():
            pltpu.sync_copy(zf, accF.at[pl.ds(zlo, 313)])
            pltpu.sync_copy(zf, accF.at[pl.ds(zlo + 313, 313)])

        pltpu.sync_copy(rowc_hbm.at[pl.ds(base, CPS)], idxs.at[pl.ds(0, CPS)])

        @pl.when(extra)
        def _():
            pltpu.sync_copy(rowc_hbm.at[base + CPS], idxs.at[CPS])

        plsc.subcore_barrier()

        def s_slot(j):
            return (pl.ds((base + j) * 128, 128), pl.ds(cid * HD, HD))

        def in_slot(j):
            return pl.ds((base + j) * 128, 128)

        def fire_load(j, b):
            pltpu.async_copy(s_hbm.at[s_slot(j)], sb.at[b], lsem.at[b])

            @pl.when(on0)
            def _():
                pltpu.async_copy(fc_hbm.at[in_slot(j)], fb.at[b], fsem.at[b])

        def wait_load(j, b):
            pltpu.make_async_copy(s_hbm.at[s_slot(j)], sb.at[b], lsem.at[b]).wait()

            @pl.when(on0)
            def _():
                pltpu.make_async_copy(fc_hbm.at[in_slot(j)], fb.at[b],
                                      fsem.at[b]).wait()

        def _scat_idx(j):
            return idxs.at[j]

        def fire_scatter(j, b):
            pltpu.async_copy(sb.at[b], accS.at[_scat_idx(j)], ssem.at[b],
                             add=True)

            @pl.when(on0)
            def _():
                pltpu.async_copy(fb.at[b], accF.at[_scat_idx(j)], tsem.at[b],
                                 add=True)

        def wait_scatter(j, b):
            pltpu.make_async_copy(sb.at[b], accS.at[_scat_idx(j)],
                                  ssem.at[b]).wait()

            @pl.when(on0)
            def _():
                pltpu.make_async_copy(fb.at[b], accF.at[_scat_idx(j)],
                                      tsem.at[b]).wait()

        fire_load(0, 0)

        @pl.loop(0, CPS, step=2)
        def _(j):
            for b in (0, 1):
                jj = j + b

                @pl.when(jj >= 1)
                def _():
                    wait_scatter(jj - 1, 1 - b)

                @pl.when((jj + 1 < CPS) | ((jj + 1 == CPS) & extra))
                def _():
                    fire_load(jj + 1, 1 - b)

                wait_load(jj, b)
                fire_scatter(jj, b)

        @pl.when(extra)
        def _():
            wait_scatter(CPS - 1, 1)
            wait_load(CPS, 0)
            fire_scatter(CPS, 0)
            wait_scatter(CPS, 0)

        @pl.when(jnp.logical_not(extra))
        def _():
            wait_scatter(CPS - 1, 1)

        plsc.subcore_barrier()
        pltpu.sync_copy(accS.at[pl.ds(sid * 626, 626)],
                        sout_hbm.at[pl.ds(sid * 626, 626), pl.ds(cid * HD, HD)])

        @pl.when(on0)
        def _():
            pltpu.sync_copy(accF.at[pl.ds(sid * 626, 626)],
                            fcout_hbm.at[pl.ds(sid * 626, 626)])

    return k(s, fc, rowc_s)


def _pre_body(h_ref, Wr_ref, Wc_ref, We2_ref, Wc1_ref, be2_ref, bc1_ref,
              T1_ref, T2_ref, Wec_ref, bec_ref):
    h = h_ref[...]
    T1_ref[...] = jnp.dot(h, Wr_ref[...], precision=_PREC)
    T2_ref[...] = jnp.dot(h, Wc_ref[...], precision=_PREC)
    Wec_ref[...] = jnp.dot(We2_ref[...], Wc1_ref[...], precision=_PREC)
    bec_ref[...] = jnp.dot(be2_ref[...], Wc1_ref[...], precision=_PREC) + bc1_ref[...]


def _node_pre(h, Wr, Wc, We2, Wc1, be2, bc1):
    NB = 1000
    grid = (N // NB,)
    return pl.pallas_call(
        _pre_body,
        grid=grid,
        in_specs=[
            pl.BlockSpec((NB, D), lambda i: (i, 0)),
            pl.BlockSpec((D, D), lambda i: (0, 0)),
            pl.BlockSpec((D, D), lambda i: (0, 0)),
            pl.BlockSpec((D, D), lambda i: (0, 0)),
            pl.BlockSpec((D, D), lambda i: (0, 0)),
            pl.BlockSpec((1, D), lambda i: (0, 0)),
            pl.BlockSpec((1, D), lambda i: (0, 0)),
        ],
        out_specs=[
            pl.BlockSpec((NB, D), lambda i: (i, 0)),
            pl.BlockSpec((NB, D), lambda i: (i, 0)),
            pl.BlockSpec((D, D), lambda i: (0, 0)),
            pl.BlockSpec((1, D), lambda i: (0, 0)),
        ],
        out_shape=[
            jax.ShapeDtypeStruct((N, D), jnp.float32),
            jax.ShapeDtypeStruct((N, D), jnp.float32),
            jax.ShapeDtypeStruct((D, D), jnp.float32),
            jax.ShapeDtypeStruct((1, D), jnp.float32),
        ],
    )(h, Wr, Wc, We2, Wc1, be2, bc1)


def _edge_body(G_ref, RIJ_ref, ea_ref,
               Ws_ref, Wa_ref, be1_ref, Wec_ref, bec_ref, Wc2_ref, bc2_ref,
               s_ref, fc_ref):
    rij = RIJ_ref[...]
    scalar = jnp.sum(rij * rij, axis=1, keepdims=True)
    u = (G_ref[...]
         + scalar * Ws_ref[...]
         + jnp.dot(ea_ref[...], Wa_ref[...], precision=_EPREC)
         + be1_ref[...])
    s = _silu(u)
    s_ref[...] = s
    t = jnp.dot(s, Wec_ref[...], precision=_EPREC) + bec_ref[...]
    cm = jnp.dot(_silu(t), Wc2_ref[...], precision=_EPREC) + bc2_ref[...]
    ones = (lax.broadcasted_iota(jnp.int32, rij.shape, 1) == 3).astype(jnp.float32)
    fc_ref[...] = rij * cm[:, 0:1] + ones


def _edge_mlp(G, RIJ, ea, Ws, Wa, be1, Wec, bec, Wc2, bc2):
    EPH = G.shape[0]
    nblk = 20
    EB = EPH // nblk
    grid = (nblk,)
    return pl.pallas_call(
        _edge_body,
        grid=grid,
        in_specs=[
            pl.BlockSpec((EB, D), lambda i: (i, 0)),
            pl.BlockSpec((EB, 16), lambda i: (i, 0)),
            pl.BlockSpec((EB, 5), lambda i: (i, 0)),
            pl.BlockSpec((1, D), lambda i: (0, 0)),
            pl.BlockSpec((5, D), lambda i: (0, 0)),
            pl.BlockSpec((1, D), lambda i: (0, 0)),
            pl.BlockSpec((D, D), lambda i: (0, 0)),
            pl.BlockSpec((1, D), lambda i: (0, 0)),
            pl.BlockSpec((D, 128), lambda i: (0, 0)),
            pl.BlockSpec((1, 128), lambda i: (0, 0)),
        ],
        out_specs=[
            pl.BlockSpec((EB, D), lambda i: (i, 0)),
            pl.BlockSpec((EB, 16), lambda i: (i, 0)),
        ],
        out_shape=[
            jax.ShapeDtypeStruct((EPH, D), jnp.float32),
            jax.ShapeDtypeStruct((EPH, 16), jnp.float32),
        ],
    )(G, RIJ, ea, Ws, Wa, be1, Wec, bec, Wc2, bc2)


def _post_body(h_ref, S0_ref, S1_ref, S2_ref, S3_ref,
               FC0_ref, FC1_ref, FC2_ref, FC3_ref, xp_ref,
               We2_ref, be2_ref, Wn1h_ref, Wn1t_ref, bn1_ref, Wn2_ref, bn2_ref,
               hn_ref, xf_ref):
    h = h_ref[...]
    S = (S0_ref[...] + S1_ref[...]) + (S2_ref[...] + S3_ref[...])
    FC = (FC0_ref[...] + FC1_ref[...]) + (FC2_ref[...] + FC3_ref[...])
    cnt = FC[:, 3:4]
    tot = jnp.dot(S, We2_ref[...], precision=_PREC) + cnt * be2_ref[...]
    a = (jnp.dot(h, Wn1h_ref[...], precision=_PREC)
         + jnp.dot(tot, Wn1t_ref[...], precision=_PREC) + bn1_ref[...])
    hn_ref[...] = h + jnp.dot(_silu(a), Wn2_ref[...], precision=_PREC) + bn2_ref[...]
    tot_f = jnp.clip(FC / jnp.maximum(cnt, 1.0), -100.0, 100.0)
    xf_ref[...] = xp_ref[...] + tot_f


def _node_post(h, Ss, FCs, xp16, We2, be2, Wn1h, Wn1t, bn1, Wn2, bn2):
    NB = 1000
    grid = (N // NB,)
    return pl.pallas_call(
        _post_body,
        grid=grid,
        in_specs=[
            pl.BlockSpec((NB, D), lambda i: (i, 0)),
            pl.BlockSpec((NB, D), lambda i: (i, 0)),
            pl.BlockSpec((NB, D), lambda i: (i, 0)),
            pl.BlockSpec((NB, D), lambda i: (i, 0)),
            pl.BlockSpec((NB, D), lambda i: (i, 0)),
            pl.BlockSpec((NB, 16), lambda i: (i, 0)),
            pl.BlockSpec((NB, 16), lambda i: (i, 0)),
            pl.BlockSpec((NB, 16), lambda i: (i, 0)),
            pl.BlockSpec((NB, 16), lambda i: (i, 0)),
            pl.BlockSpec((NB, 16), lambda i: (i, 0)),
            pl.BlockSpec((D, D), lambda i: (0, 0)),
            pl.BlockSpec((1, D), lambda i: (0, 0)),
            pl.BlockSpec((D, D), lambda i: (0, 0)),
            pl.BlockSpec((D, D), lambda i: (0, 0)),
            pl.BlockSpec((1, D), lambda i: (0, 0)),
            pl.BlockSpec((D, D), lambda i: (0, 0)),
            pl.BlockSpec((1, D), lambda i: (0, 0)),
        ],
        out_specs=[
            pl.BlockSpec((NB, D), lambda i: (i, 0)),
            pl.BlockSpec((NB, 16), lambda i: (i, 0)),
        ],
        out_shape=[
            jax.ShapeDtypeStruct((N, D), jnp.float32),
            jax.ShapeDtypeStruct((N, 16), jnp.float32),
        ],
    )(h, *Ss, *FCs, xp16, We2, be2, Wn1h, Wn1t, bn1, Wn2, bn2)


@jax.jit
def kernel(x, h, edge_index, edge_attr,
           W_e1, b_e1, W_e2, b_e2,
           W_c1, b_c1, W_c2, b_c2,
           W_n1, b_n1, W_n2, b_n2):
    row = edge_index[0].astype(jnp.int32)
    col = edge_index[1].astype(jnp.int32)
    Ws = W_e1[0:1]
    Wr = W_e1[1:1 + D]
    Wc = W_e1[1 + D:1 + 2 * D]
    Wa = W_e1[1 + 2 * D:]
    be1 = b_e1[None, :]
    be2 = b_e2[None, :]
    bc1 = b_c1[None, :]
    bc2 = jnp.zeros((1, 128), jnp.float32).at[0, 0].set(b_c2[0])
    Wc2 = jnp.zeros((D, 128), jnp.float32).at[:, 0:1].set(W_c2)
    Wn1h = W_n1[:D]
    Wn1t = W_n1[D:]
    bn1 = b_n1[None, :]
    bn2 = b_n2[None, :]

    xp16 = jnp.pad(x, ((0, 0), (0, 13)))
    T1, T2, Wec, bec = _node_pre(h, Wr, Wc, W_e2, W_c1, be2, bc1)

    rowc = row.reshape(NCHUNK, 128)
    colc = col.reshape(NCHUNK, 128)

    phases = [(0, 640), (640, 640), (1280, 640), (1920, 580)]
    souts = []
    fcouts = []
    for c0, nc in phases:
        cs = slice(c0, c0 + nc)
        es = slice(c0 * 128, (c0 + nc) * 128)
        G, RIJ = _sc_gather(T1, T2, xp16, rowc[cs], colc[cs], nc)
        s, fc = _edge_mlp(G, RIJ, edge_attr[es], Ws, Wa, be1,
                          Wec, bec, Wc2, bc2)
        so, fo = _sc_scatter(s, fc, rowc[cs], nc)
        souts.append(so[:N])
        fcouts.append(fo[:N])

    hn, xf = _node_post(h, souts, fcouts, xp16, W_e2, be2,
                        Wn1h, Wn1t, bn1, W_n2, bn2)
    return (xf[:, :3], hn)
